# Initial kernel scaffold; baseline (speedup 1.0000x reference)
import jax
import jax.numpy as jnp
from jax.experimental import pallas as pl


def kernel(x, edge_index, edge_attr, c1s1_W, c1s1_b, c1s2_W, c1s2_b, c1s3_W, c1s3_b, c2s1_W, c2s1_b, c2s2_W, c2s2_b, c2s3_W, c2s3_b, bn1_g, bn1_b, bn2_g, bn2_b, lin_W, lin_b):
    raise NotImplementedError("write your pallas kernel here")


# speedup vs baseline: 1.3660x; 1.3660x over previous
import functools
import jax
import jax.numpy as jnp
from jax.experimental import pallas as pl


def _tail_kernel(h2_ref, w_ref, b_ref, o_ref):
    logits = jnp.dot(h2_ref[...], w_ref[...], preferred_element_type=jnp.float32) + b_ref[...]
    mx = jnp.max(logits, axis=1, keepdims=True)
    e = jnp.exp(logits - mx)
    lse = jnp.log(jnp.sum(e, axis=1, keepdims=True)) + mx
    o_ref[...] = logits - lse


def kernel(x, edge_index, edge_attr, c1s1_W, c1s1_b, c1s2_W, c1s2_b, c1s3_W, c1s3_b,
           c2s1_W, c2s1_b, c2s2_W, c2s2_b, c2s3_W, c2s3_b,
           bn1_g, bn1_b, bn2_g, bn2_b, lin_W, lin_b):
    N, D = x.shape
    src, dst = edge_index[0], edge_index[1]
    ew = edge_attr
    deg = jnp.zeros((N,), x.dtype).at[src].add(ew)
    dis = jnp.where(deg > 0, jax.lax.rsqrt(jnp.where(deg > 0, deg, 1.0)), 0.0)
    norm = -dis[src] * ew * dis[dst]

    def S(y):
        return jnp.zeros_like(y).at[dst].add(norm[:, None] * y[src])

    P1 = S(x)
    P2 = S(P1)
    B0 = jnp.concatenate([c1s1_W[0], c1s2_W[0], c1s3_W[0] - c1s3_W[2]], axis=1)
    B1 = jnp.concatenate([c1s2_W[1], c1s3_W[1]], axis=1)
    B2 = 2.0 * c1s3_W[2]
    bias1 = jnp.concatenate([c1s1_b, c1s2_b, c1s3_b])
    pre = x @ B0 + bias1
    pre = pre.at[:, 400:].add(P1 @ B1)
    pre = pre.at[:, 800:].add(P2 @ B2)
    m = jnp.mean(pre, axis=0); v = jnp.mean(pre * pre, axis=0) - m * m
    scale = bn1_g * jax.lax.rsqrt(v + 1e-5); shift = bn1_b - m * scale
    h = jax.nn.relu(pre * scale + shift)

    U1 = h @ c2s2_W[1]
    U2 = h @ c2s3_W[1]
    U3 = h @ c2s3_W[2]
    HB = jnp.concatenate([c2s1_W[0], c2s2_W[0], c2s3_W[0] - c2s3_W[2]], axis=1)
    bias2 = jnp.concatenate([c2s1_b, c2s2_b, c2s3_b])
    Hb = h @ HB + bias2
    V1 = S(U1); V2 = S(U2); V3 = S(U3)
    Z = S(V3)
    add = jnp.concatenate([jnp.zeros((N, 100), x.dtype), V1, V2 + 2.0 * Z], axis=1)
    out2 = Hb + add
    m2 = jnp.mean(out2, axis=0); v2 = jnp.mean(out2 * out2, axis=0) - m2 * m2
    s2 = bn2_g * jax.lax.rsqrt(v2 + 1e-5); t2 = bn2_b - m2 * s2
    h2 = out2 * s2 + t2

    out = pl.pallas_call(
        _tail_kernel,
        out_shape=jax.ShapeDtypeStruct((N, lin_W.shape[1]), jnp.float32),
        grid=(10,),
        in_specs=[pl.BlockSpec((N // 10, 300), lambda i: (i, 0)),
                  pl.BlockSpec((300, lin_W.shape[1]), lambda i: (0, 0)),
                  pl.BlockSpec((lin_W.shape[1],), lambda i: (0,))],
        out_specs=pl.BlockSpec((N // 10, lin_W.shape[1]), lambda i: (i, 0)),
    )(h2, lin_W, lin_b)
    return out


# speedup vs baseline: 4.0099x; 2.9355x over previous
import functools
import jax
import jax.numpy as jnp
from jax import lax
from jax.experimental import pallas as pl
from jax.experimental.pallas import tpu as pltpu
from jax.experimental.pallas import tpu_sc as plsc

NC = 2
NS = 16
NW = NC * NS
CH = 128


def _sc_deg_body(n_pad, ept, src_hbm, ew_hbm, deg_hbm,
                 deg_sh, zbuf, src_v, ew_v, sem):
    cid = lax.axis_index("c")
    sid = lax.axis_index("s")
    wid = cid * NS + sid
    rows_pt = n_pad // NS

    for j in range(rows_pt // 16):
        zbuf[pl.ds(j * 16, 16)] = jnp.zeros((16,), jnp.float32)
    pltpu.sync_copy(zbuf, deg_sh.at[pl.ds(sid * rows_pt, rows_pt)])
    plsc.subcore_barrier()

    def deg_chunk(k, _):
        base = wid * ept + k * CH
        pltpu.sync_copy(src_hbm.at[pl.ds(base, CH)], src_v)
        pltpu.sync_copy(ew_hbm.at[pl.ds(base, CH)], ew_v)
        pltpu.sync_copy(ew_v, deg_sh.at[src_v], add=True)
        return 0

    lax.fori_loop(0, ept // CH, deg_chunk, 0)
    plsc.subcore_barrier()

    pltpu.sync_copy(deg_sh.at[pl.ds(sid * rows_pt, rows_pt)], zbuf)
    pltpu.sync_copy(zbuf, deg_hbm.at[pl.ds(cid * n_pad + sid * rows_pt, rows_pt)])


def _sc_deg(src_p, ew_p, n_pad):
    ept = src_p.shape[0] // NW
    rows_pt = n_pad // NS
    mesh = plsc.VectorSubcoreMesh(core_axis_name="c", subcore_axis_name="s")
    kfn = pl.kernel(
        functools.partial(_sc_deg_body, n_pad, ept),
        out_type=jax.ShapeDtypeStruct((NC * n_pad,), jnp.float32),
        mesh=mesh,
        compiler_params=pltpu.CompilerParams(needs_layout_passes=False),
        scratch_types=[
            pltpu.VMEM_SHARED((n_pad,), jnp.float32),
            pltpu.VMEM((rows_pt,), jnp.float32),
            pltpu.VMEM((CH,), jnp.int32),
            pltpu.VMEM((CH,), jnp.float32),
            pltpu.SemaphoreType.DMA,
        ],
    )
    return kfn(src_p, ew_p)


def _dis_body(a_ref, b_ref, o_ref):
    deg = a_ref[...] + b_ref[...]
    o_ref[...] = jnp.where(deg > 0, lax.rsqrt(jnp.where(deg > 0, deg, 1.0)), 0.0)


def _tc_dis(deg2, n_pad):
    a = deg2[:n_pad].reshape(n_pad // 128, 128)
    b = deg2[n_pad:].reshape(n_pad // 128, 128)
    out = pl.pallas_call(
        _dis_body,
        out_shape=jax.ShapeDtypeStruct(a.shape, jnp.float32),
    )(a, b)
    return out.reshape(n_pad)


def _sc_norm_body(n_pad, ept, src_hbm, dst_hbm, ew_hbm, dis_hbm, norm_hbm,
                  dis_v, src_v, dst_v, ew_v, nrm_v, sem):
    cid = lax.axis_index("c")
    sid = lax.axis_index("s")
    wid = cid * NS + sid

    pltpu.sync_copy(dis_hbm, dis_v)

    def nrm_chunk(k, _):
        base = wid * ept + k * CH
        pltpu.sync_copy(src_hbm.at[pl.ds(base, CH)], src_v)
        pltpu.sync_copy(dst_hbm.at[pl.ds(base, CH)], dst_v)
        pltpu.sync_copy(ew_hbm.at[pl.ds(base, CH)], ew_v)
        for j in range(CH // 16):
            sl = pl.ds(j * 16, 16)
            a = plsc.load_gather(dis_v, [src_v[sl]])
            b = plsc.load_gather(dis_v, [dst_v[sl]])
            nrm_v[sl] = -(a * ew_v[sl]) * b
        pltpu.sync_copy(nrm_v, norm_hbm.at[pl.ds(base, CH)])
        return 0

    lax.fori_loop(0, ept // CH, nrm_chunk, 0)


def _sc_norm(src_p, dst_p, ew_p, dis, n_pad):
    ept = src_p.shape[0] // NW
    mesh = plsc.VectorSubcoreMesh(core_axis_name="c", subcore_axis_name="s")
    kfn = pl.kernel(
        functools.partial(_sc_norm_body, n_pad, ept),
        out_type=jax.ShapeDtypeStruct((src_p.shape[0],), jnp.float32),
        mesh=mesh,
        compiler_params=pltpu.CompilerParams(needs_layout_passes=False),
        scratch_types=[
            pltpu.VMEM((n_pad,), jnp.float32),
            pltpu.VMEM((CH,), jnp.int32),
            pltpu.VMEM((CH,), jnp.int32),
            pltpu.VMEM((CH,), jnp.float32),
            pltpu.VMEM((CH,), jnp.float32),
            pltpu.SemaphoreType.DMA,
        ],
    )
    return kfn(src_p, dst_p, ew_p, dis)


def _sc_prop_body(n_pad, dc, ept, y_hbm, src_hbm, dst_hbm, nrm_hbm, out_hbm,
                  acc_sh, rows_v, src_v, dst_v, nrm_v, sem):
    cid = lax.axis_index("c")
    sid = lax.axis_index("s")
    wid = cid * NS + sid
    rows_pt = n_pad // NS

    def zrow(e, _):
        for j in range(dc // 16):
            rows_v[e, pl.ds(j * 16, 16)] = jnp.zeros((16,), jnp.float32)
        return 0

    lax.fori_loop(0, CH, zrow, 0)
    off = 0
    while off < rows_pt:
        sz = min(CH, rows_pt - off)
        pltpu.sync_copy(rows_v.at[pl.ds(0, sz)],
                        acc_sh.at[pl.ds(sid * rows_pt + off, sz)])
        off += sz
    plsc.subcore_barrier()

    def chunk(k, _):
        base = wid * ept + k * CH
        pltpu.sync_copy(src_hbm.at[pl.ds(base, CH)], src_v)
        pltpu.sync_copy(dst_hbm.at[pl.ds(base, CH)], dst_v)
        pltpu.sync_copy(nrm_hbm.at[pl.ds(base, CH)], nrm_v)
        pltpu.async_copy(y_hbm.at[src_v], rows_v, sem).wait()

        def scale(e, _):
            nv = plsc.load_gather(nrm_v, [jnp.zeros((16,), jnp.int32) + e])
            for j in range(dc // 16):
                sl = pl.ds(j * 16, 16)
                rows_v[e, sl] = rows_v[e, sl] * nv
            return 0

        lax.fori_loop(0, CH, scale, 0)
        pltpu.sync_copy(rows_v, acc_sh.at[dst_v], add=True)
        return 0

    lax.fori_loop(0, ept // CH, chunk, 0)
    plsc.subcore_barrier()

    off = 0
    while off < rows_pt:
        sz = min(CH, rows_pt - off)
        pltpu.sync_copy(acc_sh.at[pl.ds(sid * rows_pt + off, sz)],
                        rows_v.at[pl.ds(0, sz)])
        pltpu.sync_copy(rows_v.at[pl.ds(0, sz)],
                        out_hbm.at[pl.ds(cid * n_pad + sid * rows_pt + off, sz)])
        off += sz


def _sc_prop(y, src_p, dst_p, norm_p):
    n_nodes, dc = y.shape
    n_pad = NS * 640
    ept = src_p.shape[0] // NW
    mesh = plsc.VectorSubcoreMesh(core_axis_name="c", subcore_axis_name="s")
    kfn = pl.kernel(
        functools.partial(_sc_prop_body, n_pad, dc, ept),
        out_type=jax.ShapeDtypeStruct((NC * n_pad, dc), jnp.float32),
        mesh=mesh,
        compiler_params=pltpu.CompilerParams(needs_layout_passes=False),
        scratch_types=[
            pltpu.VMEM_SHARED((n_pad, dc), jnp.float32),
            pltpu.VMEM((CH, dc), jnp.float32),
            pltpu.VMEM((CH,), jnp.int32),
            pltpu.VMEM((CH,), jnp.int32),
            pltpu.VMEM((CH,), jnp.float32),
            pltpu.SemaphoreType.DMA,
        ],
    )
    return kfn(y, src_p, dst_p, norm_p)


def _combine_body(a_ref, b_ref, o_ref):
    o_ref[...] = a_ref[...] + b_ref[...]


def _tc_combine(a, b, bm=1000):
    n, d = a.shape
    return pl.pallas_call(
        _combine_body,
        out_shape=jax.ShapeDtypeStruct((n, d), jnp.float32),
        grid=(n // bm,),
        in_specs=[pl.BlockSpec((bm, d), lambda i: (i, 0)),
                  pl.BlockSpec((bm, d), lambda i: (i, 0))],
        out_specs=pl.BlockSpec((bm, d), lambda i: (i, 0)),
    )(a, b)


def _l1pre_body(x_ref, p1_ref, p2a_ref, p2b_ref, w_ref, b_ref, pre_ref, st_ref):
    xc = jnp.concatenate([x_ref[...], p1_ref[...], p2a_ref[...] + p2b_ref[...]],
                         axis=1)
    pre = jnp.dot(xc, w_ref[...], preferred_element_type=jnp.float32) + b_ref[...]
    pre_ref[...] = pre

    @pl.when(pl.program_id(0) == 0)
    def _():
        st_ref[...] = jnp.zeros_like(st_ref)

    st_ref[...] += jnp.stack([jnp.sum(pre, axis=0), jnp.sum(pre * pre, axis=0)])


def _tc_l1pre(x, p1, p2a, p2b, w384, bias1, bm=1000):
    n = x.shape[0]
    f = w384.shape[1]
    return pl.pallas_call(
        _l1pre_body,
        out_shape=[jax.ShapeDtypeStruct((n, f), jnp.float32),
                   jax.ShapeDtypeStruct((2, f), jnp.float32)],
        grid=(n // bm,),
        in_specs=[pl.BlockSpec((bm, 128), lambda i: (i, 0)),
                  pl.BlockSpec((bm, 128), lambda i: (i, 0)),
                  pl.BlockSpec((bm, 128), lambda i: (i, 0)),
                  pl.BlockSpec((bm, 128), lambda i: (i, 0)),
                  pl.BlockSpec((384, f), lambda i: (0, 0)),
                  pl.BlockSpec((f,), lambda i: (0,))],
        out_specs=[pl.BlockSpec((bm, f), lambda i: (i, 0)),
                   pl.BlockSpec((2, f), lambda i: (0, 0))],
    )(x, p1, p2a, p2b, w384, bias1)


def _l1post_body(n_rows, pre_ref, st_ref, g_ref, bb_ref, wu1_ref, wu2_ref,
                 wu3_ref, whb_ref, b2_ref, u1_ref, u2_ref, u3_ref, hb_ref):
    m = st_ref[0, :] / n_rows
    var = st_ref[1, :] / n_rows - m * m
    scale = g_ref[...] * lax.rsqrt(var + 1e-5)
    shift = bb_ref[...] - m * scale
    h = jnp.maximum(pre_ref[...] * scale + shift, 0.0)
    u1_ref[...] = jnp.dot(h, wu1_ref[...], preferred_element_type=jnp.float32)
    u2_ref[...] = jnp.dot(h, wu2_ref[...], preferred_element_type=jnp.float32)
    u3_ref[...] = jnp.dot(h, wu3_ref[...], preferred_element_type=jnp.float32)
    hb_ref[...] = jnp.dot(h, whb_ref[...], preferred_element_type=jnp.float32) + b2_ref[...]


def _tc_l1post(pre, stats, g, b, wu1, wu2, wu3, whb, bias2, bm=1000):
    n, f = pre.shape
    du = wu1.shape[1]
    return pl.pallas_call(
        functools.partial(_l1post_body, float(n)),
        out_shape=[jax.ShapeDtypeStruct((n, du), jnp.float32),
                   jax.ShapeDtypeStruct((n, du), jnp.float32),
                   jax.ShapeDtypeStruct((n, du), jnp.float32),
                   jax.ShapeDtypeStruct((n, 300), jnp.float32)],
        grid=(n // bm,),
        in_specs=[pl.BlockSpec((bm, f), lambda i: (i, 0)),
                  pl.BlockSpec((2, f), lambda i: (0, 0)),
                  pl.BlockSpec((f,), lambda i: (0,)),
                  pl.BlockSpec((f,), lambda i: (0,)),
                  pl.BlockSpec((f, du), lambda i: (0, 0)),
                  pl.BlockSpec((f, du), lambda i: (0, 0)),
                  pl.BlockSpec((f, du), lambda i: (0, 0)),
                  pl.BlockSpec((f, 300), lambda i: (0, 0)),
                  pl.BlockSpec((300,), lambda i: (0,))],
        out_specs=[pl.BlockSpec((bm, du), lambda i: (i, 0)),
                   pl.BlockSpec((bm, du), lambda i: (i, 0)),
                   pl.BlockSpec((bm, du), lambda i: (i, 0)),
                   pl.BlockSpec((bm, 300), lambda i: (i, 0))],
    )(pre, stats, g, b, wu1, wu2, wu3, whb, bias2)


def _fpre_body(hb_ref, v1a_ref, v1b_ref, v2a_ref, v2b_ref, za_ref, zb_ref,
               o_ref, st_ref):
    bm = hb_ref.shape[0]
    v1 = (v1a_ref[...] + v1b_ref[...])[:, :100]
    v23 = (v2a_ref[...] + v2b_ref[...] + 2.0 * (za_ref[...] + zb_ref[...]))[:, :100]
    add = jnp.concatenate([jnp.zeros((bm, 100), jnp.float32), v1, v23], axis=1)
    o = hb_ref[...] + add
    o_ref[...] = o

    @pl.when(pl.program_id(0) == 0)
    def _():
        st_ref[...] = jnp.zeros_like(st_ref)

    st_ref[...] += jnp.stack([jnp.sum(o, axis=0), jnp.sum(o * o, axis=0)])


def _tc_fpre(hb, v1a, v1b, v2a, v2b, za, zb, bm=1000):
    n = hb.shape[0]
    du = v1a.shape[1]
    return pl.pallas_call(
        _fpre_body,
        out_shape=[jax.ShapeDtypeStruct((n, 300), jnp.float32),
                   jax.ShapeDtypeStruct((2, 300), jnp.float32)],
        grid=(n // bm,),
        in_specs=[pl.BlockSpec((bm, 300), lambda i: (i, 0))] +
                 [pl.BlockSpec((bm, du), lambda i: (i, 0))] * 6,
        out_specs=[pl.BlockSpec((bm, 300), lambda i: (i, 0)),
                   pl.BlockSpec((2, 300), lambda i: (0, 0))],
    )(hb, v1a, v1b, v2a, v2b, za, zb)


def _fpost_body(n_rows, o2_ref, st_ref, g_ref, bb_ref, lw_ref, lb_ref, o_ref):
    m = st_ref[0, :] / n_rows
    var = st_ref[1, :] / n_rows - m * m
    scale = g_ref[...] * lax.rsqrt(var + 1e-5)
    shift = bb_ref[...] - m * scale
    h2 = o2_ref[...] * scale + shift
    logits = jnp.dot(h2, lw_ref[...], preferred_element_type=jnp.float32) + lb_ref[...]
    mx = jnp.max(logits, axis=1, keepdims=True)
    lse = jnp.log(jnp.sum(jnp.exp(logits - mx), axis=1, keepdims=True)) + mx
    o_ref[...] = logits - lse


def _tc_fpost(out2, stats, g, b, lw, lb, bm=1000):
    n = out2.shape[0]
    k = lw.shape[1]
    return pl.pallas_call(
        functools.partial(_fpost_body, float(n)),
        out_shape=jax.ShapeDtypeStruct((n, k), jnp.float32),
        grid=(n // bm,),
        in_specs=[pl.BlockSpec((bm, 300), lambda i: (i, 0)),
                  pl.BlockSpec((2, 300), lambda i: (0, 0)),
                  pl.BlockSpec((300,), lambda i: (0,)),
                  pl.BlockSpec((300,), lambda i: (0,)),
                  pl.BlockSpec((300, k), lambda i: (0, 0)),
                  pl.BlockSpec((k,), lambda i: (0,))],
        out_specs=pl.BlockSpec((bm, k), lambda i: (i, 0)),
    )(out2, stats, g, b, lw, lb)


def kernel(x, edge_index, edge_attr, c1s1_W, c1s1_b, c1s2_W, c1s2_b, c1s3_W,
           c1s3_b, c2s1_W, c2s1_b, c2s2_W, c2s2_b, c2s3_W, c2s3_b,
           bn1_g, bn1_b, bn2_g, bn2_b, lin_W, lin_b):
    n, d = x.shape
    e = edge_index.shape[1]
    ept = -(-e // (NW * CH)) * CH
    e_pad = ept * NW

    src = jnp.pad(edge_index[0], (0, e_pad - e))
    dst = jnp.pad(edge_index[1], (0, e_pad - e))
    ew = jnp.pad(edge_attr, (0, e_pad - e))

    n_pad = NS * 640
    deg2 = _sc_deg(src, ew, n_pad)
    dis = _tc_dis(deg2, n_pad)
    norm = _sc_norm(src, dst, ew, dis, n_pad)

    p1p = _sc_prop(x, src, dst, norm)
    p1 = _tc_combine(p1p[:n], p1p[n_pad:n_pad + n])
    p2p = _sc_prop(p1, src, dst, norm)

    z128 = jnp.zeros((128, 400), jnp.float32)
    w384 = jnp.concatenate([
        jnp.concatenate([c1s1_W[0], c1s2_W[0], c1s3_W[0] - c1s3_W[2]], axis=1),
        jnp.concatenate([z128, c1s2_W[1], c1s3_W[1]], axis=1),
        jnp.concatenate([z128, z128, 2.0 * c1s3_W[2]], axis=1)], axis=0)
    bias1 = jnp.concatenate([c1s1_b, c1s2_b, c1s3_b])
    pre, stats1 = _tc_l1pre(x, p1, p2p[:n], p2p[n_pad:n_pad + n], w384, bias1)

    def pad128(w):
        return jnp.pad(w, ((0, 0), (0, 28)))
    whb = jnp.concatenate([c2s1_W[0], c2s2_W[0], c2s3_W[0] - c2s3_W[2]], axis=1)
    bias2 = jnp.concatenate([c2s1_b, c2s2_b, c2s3_b])
    u1, u2, u3, hb = _tc_l1post(pre, stats1, bn1_g, bn1_b,
                                pad128(c2s2_W[1]), pad128(c2s3_W[1]),
                                pad128(c2s3_W[2]), whb, bias2)

    v1p = _sc_prop(u1, src, dst, norm)
    v2p = _sc_prop(u2, src, dst, norm)
    v3p = _sc_prop(u3, src, dst, norm)
    v3 = _tc_combine(v3p[:n], v3p[n_pad:n_pad + n])
    zp = _sc_prop(v3, src, dst, norm)

    out2, stats2 = _tc_fpre(hb, v1p[:n], v1p[n_pad:n_pad + n],
                            v2p[:n], v2p[n_pad:n_pad + n],
                            zp[:n], zp[n_pad:n_pad + n])
    return _tc_fpost(out2, stats2, bn2_g, bn2_b, lin_W, lin_b)


# speedup vs baseline: 5.1171x; 1.2761x over previous
import functools
import jax
import jax.numpy as jnp
from jax import lax
from jax.experimental import pallas as pl
from jax.experimental.pallas import tpu as pltpu
from jax.experimental.pallas import tpu_sc as plsc

NC = 2
NS = 16
NW = NC * NS
CH = 128


def _sc_deg_body(n_pad, ept, src_hbm, ew_hbm, deg_hbm,
                 deg_sh, zbuf, src_v, ew_v, sem):
    cid = lax.axis_index("c")
    sid = lax.axis_index("s")
    wid = cid * NS + sid
    rows_pt = n_pad // NS

    for j in range(rows_pt // 16):
        zbuf[pl.ds(j * 16, 16)] = jnp.zeros((16,), jnp.float32)
    pltpu.sync_copy(zbuf, deg_sh.at[pl.ds(sid * rows_pt, rows_pt)])
    plsc.subcore_barrier()

    def deg_chunk(k, _):
        base = wid * ept + k * CH
        pltpu.sync_copy(src_hbm.at[pl.ds(base, CH)], src_v)
        pltpu.sync_copy(ew_hbm.at[pl.ds(base, CH)], ew_v)
        pltpu.sync_copy(ew_v, deg_sh.at[src_v], add=True)
        return 0

    lax.fori_loop(0, ept // CH, deg_chunk, 0)
    plsc.subcore_barrier()

    pltpu.sync_copy(deg_sh.at[pl.ds(sid * rows_pt, rows_pt)], zbuf)
    pltpu.sync_copy(zbuf, deg_hbm.at[pl.ds(cid * n_pad + sid * rows_pt, rows_pt)])


def _sc_deg(src_p, ew_p, n_pad):
    ept = src_p.shape[0] // NW
    rows_pt = n_pad // NS
    mesh = plsc.VectorSubcoreMesh(core_axis_name="c", subcore_axis_name="s")
    kfn = pl.kernel(
        functools.partial(_sc_deg_body, n_pad, ept),
        out_type=jax.ShapeDtypeStruct((NC * n_pad,), jnp.float32),
        mesh=mesh,
        compiler_params=pltpu.CompilerParams(needs_layout_passes=False),
        scratch_types=[
            pltpu.VMEM_SHARED((n_pad,), jnp.float32),
            pltpu.VMEM((rows_pt,), jnp.float32),
            pltpu.VMEM((CH,), jnp.int32),
            pltpu.VMEM((CH,), jnp.float32),
            pltpu.SemaphoreType.DMA,
        ],
    )
    return kfn(src_p, ew_p)


def _dis_body(a_ref, b_ref, o_ref):
    deg = a_ref[...] + b_ref[...]
    o_ref[...] = jnp.where(deg > 0, lax.rsqrt(jnp.where(deg > 0, deg, 1.0)), 0.0)


def _tc_dis(deg2, n_pad):
    a = deg2[:n_pad].reshape(n_pad // 128, 128)
    b = deg2[n_pad:].reshape(n_pad // 128, 128)
    out = pl.pallas_call(
        _dis_body,
        out_shape=jax.ShapeDtypeStruct(a.shape, jnp.float32),
    )(a, b)
    return out.reshape(n_pad)


def _sc_norm_body(n_pad, ept, src_hbm, dst_hbm, ew_hbm, dis_hbm, norm_hbm,
                  dis_v, src_v, dst_v, ew_v, nrm_v, sem):
    cid = lax.axis_index("c")
    sid = lax.axis_index("s")
    wid = cid * NS + sid

    pltpu.sync_copy(dis_hbm, dis_v)

    def nrm_chunk(k, _):
        base = wid * ept + k * CH
        pltpu.sync_copy(src_hbm.at[pl.ds(base, CH)], src_v)
        pltpu.sync_copy(dst_hbm.at[pl.ds(base, CH)], dst_v)
        pltpu.sync_copy(ew_hbm.at[pl.ds(base, CH)], ew_v)
        for j in range(CH // 16):
            sl = pl.ds(j * 16, 16)
            a = plsc.load_gather(dis_v, [src_v[sl]])
            b = plsc.load_gather(dis_v, [dst_v[sl]])
            nrm_v[sl] = -(a * ew_v[sl]) * b
        pltpu.sync_copy(nrm_v, norm_hbm.at[pl.ds(base, CH)])
        return 0

    lax.fori_loop(0, ept // CH, nrm_chunk, 0)


def _sc_norm(src_p, dst_p, ew_p, dis, n_pad):
    ept = src_p.shape[0] // NW
    mesh = plsc.VectorSubcoreMesh(core_axis_name="c", subcore_axis_name="s")
    kfn = pl.kernel(
        functools.partial(_sc_norm_body, n_pad, ept),
        out_type=jax.ShapeDtypeStruct((src_p.shape[0],), jnp.float32),
        mesh=mesh,
        compiler_params=pltpu.CompilerParams(needs_layout_passes=False),
        scratch_types=[
            pltpu.VMEM((n_pad,), jnp.float32),
            pltpu.VMEM((CH,), jnp.int32),
            pltpu.VMEM((CH,), jnp.int32),
            pltpu.VMEM((CH,), jnp.float32),
            pltpu.VMEM((CH,), jnp.float32),
            pltpu.SemaphoreType.DMA,
        ],
    )
    return kfn(src_p, dst_p, ew_p, dis)


def _sc_prop_body(n_pad, dc, ept, y_hbm, src2_hbm, dst2_hbm, nrm_hbm, out_hbm,
                  acc_sh, rows_a, rows_b, src2_v, dst2_v, nrm_v, sem_a, sem_b):
    cid = lax.axis_index("c")
    sid = lax.axis_index("s")
    wid = cid * NS + sid
    rows_pt = n_pad // NS
    nch = ept // CH

    pltpu.sync_copy(src2_hbm.at[pl.ds(wid * nch, nch)], src2_v)
    pltpu.sync_copy(dst2_hbm.at[pl.ds(wid * nch, nch)], dst2_v)
    pltpu.sync_copy(nrm_hbm.at[pl.ds(wid * ept, ept)], nrm_v)

    def zrow(e, _):
        for j in range(dc // 16):
            rows_a[e, pl.ds(j * 16, 16)] = jnp.zeros((16,), jnp.float32)
        return 0

    lax.fori_loop(0, CH, zrow, 0)
    off = 0
    while off < rows_pt:
        sz = min(CH, rows_pt - off)
        pltpu.sync_copy(rows_a.at[pl.ds(0, sz)],
                        acc_sh.at[pl.ds(sid * rows_pt + off, sz)])
        off += sz
    plsc.subcore_barrier()

    def scale_scatter(k, rows):
        def scale(e, _):
            nv = plsc.load_gather(nrm_v, [jnp.zeros((16,), jnp.int32) + k * CH + e])
            for j in range(dc // 16):
                sl = pl.ds(j * 16, 16)
                rows[e, sl] = rows[e, sl] * nv
            return 0

        lax.fori_loop(0, CH, scale, 0)
        pltpu.sync_copy(rows, acc_sh.at[dst2_v.at[k]], add=True)

    pltpu.async_copy(y_hbm.at[src2_v.at[0]], rows_a, sem_a)

    def pair(g, _):
        k0 = 2 * g
        pltpu.make_async_copy(y_hbm.at[src2_v.at[k0]], rows_a, sem_a).wait()
        pltpu.async_copy(y_hbm.at[src2_v.at[k0 + 1]], rows_b, sem_b)
        scale_scatter(k0, rows_a)
        pltpu.make_async_copy(y_hbm.at[src2_v.at[k0 + 1]], rows_b, sem_b).wait()

        @pl.when(g + 1 < nch // 2)
        def _():
            pltpu.async_copy(y_hbm.at[src2_v.at[k0 + 2]], rows_a, sem_a)

        scale_scatter(k0 + 1, rows_b)
        return 0

    lax.fori_loop(0, nch // 2, pair, 0)
    plsc.subcore_barrier()

    off = 0
    while off < rows_pt:
        sz = min(CH, rows_pt - off)
        pltpu.sync_copy(acc_sh.at[pl.ds(sid * rows_pt + off, sz)],
                        rows_a.at[pl.ds(0, sz)])
        pltpu.sync_copy(rows_a.at[pl.ds(0, sz)],
                        out_hbm.at[pl.ds(cid * n_pad + sid * rows_pt + off, sz)])
        off += sz


def _sc_prop(y, src2, dst2, norm_p):
    n_nodes, dc = y.shape
    n_pad = NS * 640
    ept = norm_p.shape[0] // NW
    assert (ept // CH) % 2 == 0
    mesh = plsc.VectorSubcoreMesh(core_axis_name="c", subcore_axis_name="s")
    kfn = pl.kernel(
        functools.partial(_sc_prop_body, n_pad, dc, ept),
        out_type=jax.ShapeDtypeStruct((NC * n_pad, dc), jnp.float32),
        mesh=mesh,
        compiler_params=pltpu.CompilerParams(needs_layout_passes=False),
        scratch_types=[
            pltpu.VMEM_SHARED((n_pad, dc), jnp.float32),
            pltpu.VMEM((CH, dc), jnp.float32),
            pltpu.VMEM((CH, dc), jnp.float32),
            pltpu.VMEM((ept // CH, CH), jnp.int32),
            pltpu.VMEM((ept // CH, CH), jnp.int32),
            pltpu.VMEM((ept,), jnp.float32),
            pltpu.SemaphoreType.DMA,
            pltpu.SemaphoreType.DMA,
        ],
    )
    return kfn(y, src2, dst2, norm_p)


def _combine_body(a_ref, b_ref, o_ref):
    o_ref[...] = a_ref[...] + b_ref[...]


def _tc_combine(a, b, bm=1000):
    n, d = a.shape
    return pl.pallas_call(
        _combine_body,
        out_shape=jax.ShapeDtypeStruct((n, d), jnp.float32),
        grid=(n // bm,),
        in_specs=[pl.BlockSpec((bm, d), lambda i: (i, 0)),
                  pl.BlockSpec((bm, d), lambda i: (i, 0))],
        out_specs=pl.BlockSpec((bm, d), lambda i: (i, 0)),
    )(a, b)


def _l1pre_body(x_ref, p1_ref, p2a_ref, p2b_ref, w_ref, b_ref, pre_ref, st_ref):
    xc = jnp.concatenate([x_ref[...], p1_ref[...], p2a_ref[...] + p2b_ref[...]],
                         axis=1)
    pre = jnp.dot(xc, w_ref[...], preferred_element_type=jnp.float32) + b_ref[...]
    pre_ref[...] = pre

    @pl.when(pl.program_id(0) == 0)
    def _():
        st_ref[...] = jnp.zeros_like(st_ref)

    st_ref[...] += jnp.stack([jnp.sum(pre, axis=0), jnp.sum(pre * pre, axis=0)])


def _tc_l1pre(x, p1, p2a, p2b, w384, bias1, bm=1000):
    n = x.shape[0]
    f = w384.shape[1]
    return pl.pallas_call(
        _l1pre_body,
        out_shape=[jax.ShapeDtypeStruct((n, f), jnp.float32),
                   jax.ShapeDtypeStruct((2, f), jnp.float32)],
        grid=(n // bm,),
        in_specs=[pl.BlockSpec((bm, 128), lambda i: (i, 0)),
                  pl.BlockSpec((bm, 128), lambda i: (i, 0)),
                  pl.BlockSpec((bm, 128), lambda i: (i, 0)),
                  pl.BlockSpec((bm, 128), lambda i: (i, 0)),
                  pl.BlockSpec((384, f), lambda i: (0, 0)),
                  pl.BlockSpec((f,), lambda i: (0,))],
        out_specs=[pl.BlockSpec((bm, f), lambda i: (i, 0)),
                   pl.BlockSpec((2, f), lambda i: (0, 0))],
    )(x, p1, p2a, p2b, w384, bias1)


def _l1post_body(n_rows, pre_ref, st_ref, g_ref, bb_ref, wu1_ref, wu2_ref,
                 wu3_ref, whb_ref, b2_ref, u1_ref, u2_ref, u3_ref, hb_ref):
    m = st_ref[0, :] / n_rows
    var = st_ref[1, :] / n_rows - m * m
    scale = g_ref[...] * lax.rsqrt(var + 1e-5)
    shift = bb_ref[...] - m * scale
    h = jnp.maximum(pre_ref[...] * scale + shift, 0.0)
    u1_ref[...] = jnp.dot(h, wu1_ref[...], preferred_element_type=jnp.float32)
    u2_ref[...] = jnp.dot(h, wu2_ref[...], preferred_element_type=jnp.float32)
    u3_ref[...] = jnp.dot(h, wu3_ref[...], preferred_element_type=jnp.float32)
    hb_ref[...] = jnp.dot(h, whb_ref[...], preferred_element_type=jnp.float32) + b2_ref[...]


def _tc_l1post(pre, stats, g, b, wu1, wu2, wu3, whb, bias2, bm=1000):
    n, f = pre.shape
    du = wu1.shape[1]
    return pl.pallas_call(
        functools.partial(_l1post_body, float(n)),
        out_shape=[jax.ShapeDtypeStruct((n, du), jnp.float32),
                   jax.ShapeDtypeStruct((n, du), jnp.float32),
                   jax.ShapeDtypeStruct((n, du), jnp.float32),
                   jax.ShapeDtypeStruct((n, 300), jnp.float32)],
        grid=(n // bm,),
        in_specs=[pl.BlockSpec((bm, f), lambda i: (i, 0)),
                  pl.BlockSpec((2, f), lambda i: (0, 0)),
                  pl.BlockSpec((f,), lambda i: (0,)),
                  pl.BlockSpec((f,), lambda i: (0,)),
                  pl.BlockSpec((f, du), lambda i: (0, 0)),
                  pl.BlockSpec((f, du), lambda i: (0, 0)),
                  pl.BlockSpec((f, du), lambda i: (0, 0)),
                  pl.BlockSpec((f, 300), lambda i: (0, 0)),
                  pl.BlockSpec((300,), lambda i: (0,))],
        out_specs=[pl.BlockSpec((bm, du), lambda i: (i, 0)),
                   pl.BlockSpec((bm, du), lambda i: (i, 0)),
                   pl.BlockSpec((bm, du), lambda i: (i, 0)),
                   pl.BlockSpec((bm, 300), lambda i: (i, 0))],
    )(pre, stats, g, b, wu1, wu2, wu3, whb, bias2)


def _fpre_body(hb_ref, v1a_ref, v1b_ref, v2a_ref, v2b_ref, za_ref, zb_ref,
               o_ref, st_ref):
    bm = hb_ref.shape[0]
    v1 = (v1a_ref[...] + v1b_ref[...])[:, :100]
    v23 = (v2a_ref[...] + v2b_ref[...] + 2.0 * (za_ref[...] + zb_ref[...]))[:, :100]
    add = jnp.concatenate([jnp.zeros((bm, 100), jnp.float32), v1, v23], axis=1)
    o = hb_ref[...] + add
    o_ref[...] = o

    @pl.when(pl.program_id(0) == 0)
    def _():
        st_ref[...] = jnp.zeros_like(st_ref)

    st_ref[...] += jnp.stack([jnp.sum(o, axis=0), jnp.sum(o * o, axis=0)])


def _tc_fpre(hb, v1a, v1b, v2a, v2b, za, zb, bm=1000):
    n = hb.shape[0]
    du = v1a.shape[1]
    return pl.pallas_call(
        _fpre_body,
        out_shape=[jax.ShapeDtypeStruct((n, 300), jnp.float32),
                   jax.ShapeDtypeStruct((2, 300), jnp.float32)],
        grid=(n // bm,),
        in_specs=[pl.BlockSpec((bm, 300), lambda i: (i, 0))] +
                 [pl.BlockSpec((bm, du), lambda i: (i, 0))] * 6,
        out_specs=[pl.BlockSpec((bm, 300), lambda i: (i, 0)),
                   pl.BlockSpec((2, 300), lambda i: (0, 0))],
    )(hb, v1a, v1b, v2a, v2b, za, zb)


def _fpost_body(n_rows, o2_ref, st_ref, g_ref, bb_ref, lw_ref, lb_ref, o_ref):
    m = st_ref[0, :] / n_rows
    var = st_ref[1, :] / n_rows - m * m
    scale = g_ref[...] * lax.rsqrt(var + 1e-5)
    shift = bb_ref[...] - m * scale
    h2 = o2_ref[...] * scale + shift
    logits = jnp.dot(h2, lw_ref[...], preferred_element_type=jnp.float32) + lb_ref[...]
    mx = jnp.max(logits, axis=1, keepdims=True)
    lse = jnp.log(jnp.sum(jnp.exp(logits - mx), axis=1, keepdims=True)) + mx
    o_ref[...] = logits - lse


def _tc_fpost(out2, stats, g, b, lw, lb, bm=1000):
    n = out2.shape[0]
    k = lw.shape[1]
    return pl.pallas_call(
        functools.partial(_fpost_body, float(n)),
        out_shape=jax.ShapeDtypeStruct((n, k), jnp.float32),
        grid=(n // bm,),
        in_specs=[pl.BlockSpec((bm, 300), lambda i: (i, 0)),
                  pl.BlockSpec((2, 300), lambda i: (0, 0)),
                  pl.BlockSpec((300,), lambda i: (0,)),
                  pl.BlockSpec((300,), lambda i: (0,)),
                  pl.BlockSpec((300, k), lambda i: (0, 0)),
                  pl.BlockSpec((k,), lambda i: (0,))],
        out_specs=pl.BlockSpec((bm, k), lambda i: (i, 0)),
    )(out2, stats, g, b, lw, lb)


def kernel(x, edge_index, edge_attr, c1s1_W, c1s1_b, c1s2_W, c1s2_b, c1s3_W,
           c1s3_b, c2s1_W, c2s1_b, c2s2_W, c2s2_b, c2s3_W, c2s3_b,
           bn1_g, bn1_b, bn2_g, bn2_b, lin_W, lin_b):
    n, d = x.shape
    e = edge_index.shape[1]
    ept = -(-e // (NW * CH)) * CH
    e_pad = ept * NW

    src = jnp.pad(edge_index[0], (0, e_pad - e))
    dst = jnp.pad(edge_index[1], (0, e_pad - e))
    ew = jnp.pad(edge_attr, (0, e_pad - e))

    n_pad = NS * 640
    src2 = src.reshape(-1, CH)
    dst2 = dst.reshape(-1, CH)
    deg2 = _sc_deg(src, ew, n_pad)
    dis = _tc_dis(deg2, n_pad)
    norm = _sc_norm(src, dst, ew, dis, n_pad)

    p1p = _sc_prop(x, src2, dst2, norm)
    p1 = _tc_combine(p1p[:n], p1p[n_pad:n_pad + n])
    p2p = _sc_prop(p1, src2, dst2, norm)

    z128 = jnp.zeros((128, 400), jnp.float32)
    w384 = jnp.concatenate([
        jnp.concatenate([c1s1_W[0], c1s2_W[0], c1s3_W[0] - c1s3_W[2]], axis=1),
        jnp.concatenate([z128, c1s2_W[1], c1s3_W[1]], axis=1),
        jnp.concatenate([z128, z128, 2.0 * c1s3_W[2]], axis=1)], axis=0)
    bias1 = jnp.concatenate([c1s1_b, c1s2_b, c1s3_b])
    pre, stats1 = _tc_l1pre(x, p1, p2p[:n], p2p[n_pad:n_pad + n], w384, bias1)

    def pad128(w):
        return jnp.pad(w, ((0, 0), (0, 28)))
    whb = jnp.concatenate([c2s1_W[0], c2s2_W[0], c2s3_W[0] - c2s3_W[2]], axis=1)
    bias2 = jnp.concatenate([c2s1_b, c2s2_b, c2s3_b])
    u1, u2, u3, hb = _tc_l1post(pre, stats1, bn1_g, bn1_b,
                                pad128(c2s2_W[1]), pad128(c2s3_W[1]),
                                pad128(c2s3_W[2]), whb, bias2)

    v1p = _sc_prop(u1, src2, dst2, norm)
    v2p = _sc_prop(u2, src2, dst2, norm)
    v3p = _sc_prop(u3, src2, dst2, norm)
    v3 = _tc_combine(v3p[:n], v3p[n_pad:n_pad + n])
    zp = _sc_prop(v3, src2, dst2, norm)

    out2, stats2 = _tc_fpre(hb, v1p[:n], v1p[n_pad:n_pad + n],
                            v2p[:n], v2p[n_pad:n_pad + n],
                            zp[:n], zp[n_pad:n_pad + n])
    return _tc_fpost(out2, stats2, bn2_g, bn2_b, lin_W, lin_b)


# speedup vs baseline: 5.1721x; 1.0107x over previous
import functools
import jax
import jax.numpy as jnp
from jax import lax
from jax.experimental import pallas as pl
from jax.experimental.pallas import tpu as pltpu
from jax.experimental.pallas import tpu_sc as plsc

NC = 2
NS = 16
NW = NC * NS
CH = 128


def _sc_deg_body(n_pad, ept, src_hbm, ew_hbm, deg_hbm,
                 deg_sh, zbuf, src_v, ew_v, sem):
    cid = lax.axis_index("c")
    sid = lax.axis_index("s")
    wid = cid * NS + sid
    rows_pt = n_pad // NS

    for j in range(rows_pt // 16):
        zbuf[pl.ds(j * 16, 16)] = jnp.zeros((16,), jnp.float32)
    pltpu.sync_copy(zbuf, deg_sh.at[pl.ds(sid * rows_pt, rows_pt)])
    plsc.subcore_barrier()

    def deg_chunk(k, _):
        base = wid * ept + k * CH
        pltpu.sync_copy(src_hbm.at[pl.ds(base, CH)], src_v)
        pltpu.sync_copy(ew_hbm.at[pl.ds(base, CH)], ew_v)
        pltpu.sync_copy(ew_v, deg_sh.at[src_v], add=True)
        return 0

    lax.fori_loop(0, ept // CH, deg_chunk, 0)
    plsc.subcore_barrier()

    pltpu.sync_copy(deg_sh.at[pl.ds(sid * rows_pt, rows_pt)], zbuf)
    pltpu.sync_copy(zbuf, deg_hbm.at[pl.ds(cid * n_pad + sid * rows_pt, rows_pt)])


def _sc_deg(src_p, ew_p, n_pad):
    ept = src_p.shape[0] // NW
    rows_pt = n_pad // NS
    mesh = plsc.VectorSubcoreMesh(core_axis_name="c", subcore_axis_name="s")
    kfn = pl.kernel(
        functools.partial(_sc_deg_body, n_pad, ept),
        out_type=jax.ShapeDtypeStruct((NC * n_pad,), jnp.float32),
        mesh=mesh,
        compiler_params=pltpu.CompilerParams(needs_layout_passes=False),
        scratch_types=[
            pltpu.VMEM_SHARED((n_pad,), jnp.float32),
            pltpu.VMEM((rows_pt,), jnp.float32),
            pltpu.VMEM((CH,), jnp.int32),
            pltpu.VMEM((CH,), jnp.float32),
            pltpu.SemaphoreType.DMA,
        ],
    )
    return kfn(src_p, ew_p)


def _dis_body(a_ref, b_ref, o_ref):
    deg = a_ref[...] + b_ref[...]
    o_ref[...] = jnp.where(deg > 0, lax.rsqrt(jnp.where(deg > 0, deg, 1.0)), 0.0)


def _tc_dis(deg2, n_pad):
    a = deg2[:n_pad].reshape(n_pad // 128, 128)
    b = deg2[n_pad:].reshape(n_pad // 128, 128)
    out = pl.pallas_call(
        _dis_body,
        out_shape=jax.ShapeDtypeStruct(a.shape, jnp.float32),
    )(a, b)
    return out.reshape(n_pad)


def _sc_norm_body(n_pad, ept, src_hbm, dst_hbm, ew_hbm, dis_hbm, norm_hbm,
                  dis_v, src_v, dst_v, ew_v, nrm_v, sem):
    cid = lax.axis_index("c")
    sid = lax.axis_index("s")
    wid = cid * NS + sid

    pltpu.sync_copy(dis_hbm, dis_v)

    def nrm_chunk(k, _):
        base = wid * ept + k * CH
        pltpu.sync_copy(src_hbm.at[pl.ds(base, CH)], src_v)
        pltpu.sync_copy(dst_hbm.at[pl.ds(base, CH)], dst_v)
        pltpu.sync_copy(ew_hbm.at[pl.ds(base, CH)], ew_v)
        for j in range(CH // 16):
            sl = pl.ds(j * 16, 16)
            a = plsc.load_gather(dis_v, [src_v[sl]])
            b = plsc.load_gather(dis_v, [dst_v[sl]])
            nrm_v[sl] = -(a * ew_v[sl]) * b
        pltpu.sync_copy(nrm_v, norm_hbm.at[pl.ds(base, CH)])
        return 0

    lax.fori_loop(0, ept // CH, nrm_chunk, 0)


def _sc_norm(src_p, dst_p, ew_p, dis, n_pad):
    ept = src_p.shape[0] // NW
    mesh = plsc.VectorSubcoreMesh(core_axis_name="c", subcore_axis_name="s")
    kfn = pl.kernel(
        functools.partial(_sc_norm_body, n_pad, ept),
        out_type=jax.ShapeDtypeStruct((src_p.shape[0],), jnp.float32),
        mesh=mesh,
        compiler_params=pltpu.CompilerParams(needs_layout_passes=False),
        scratch_types=[
            pltpu.VMEM((n_pad,), jnp.float32),
            pltpu.VMEM((CH,), jnp.int32),
            pltpu.VMEM((CH,), jnp.int32),
            pltpu.VMEM((CH,), jnp.float32),
            pltpu.VMEM((CH,), jnp.float32),
            pltpu.SemaphoreType.DMA,
        ],
    )
    return kfn(src_p, dst_p, ew_p, dis)


def _sc_prop_body(n_pad, dc, ept, y_hbm, src2_hbm, dst2_hbm, nrm_hbm, out_hbm,
                  acc_sh, rows_a, rows_b, src2_v, dst2_v, nrm_v, sem_a, sem_b):
    cid = lax.axis_index("c")
    sid = lax.axis_index("s")
    wid = cid * NS + sid
    rows_pt = n_pad // NS
    nch = ept // CH

    pltpu.sync_copy(src2_hbm.at[pl.ds(wid * nch, nch)], src2_v)
    pltpu.sync_copy(dst2_hbm.at[pl.ds(wid * nch, nch)], dst2_v)
    pltpu.sync_copy(nrm_hbm.at[pl.ds(wid * ept, ept)], nrm_v)

    def zrow(e, _):
        for j in range(dc // 16):
            rows_a[e, pl.ds(j * 16, 16)] = jnp.zeros((16,), jnp.float32)
        return 0

    lax.fori_loop(0, CH, zrow, 0)
    off = 0
    while off < rows_pt:
        sz = min(CH, rows_pt - off)
        pltpu.sync_copy(rows_a.at[pl.ds(0, sz)],
                        acc_sh.at[pl.ds(sid * rows_pt + off, sz)])
        off += sz
    plsc.subcore_barrier()

    def scale_scatter(k, rows):
        @plsc.parallel_loop(0, CH, unroll=4)
        def scale(e):
            nv = plsc.load_gather(nrm_v, [jnp.zeros((16,), jnp.int32) + k * CH + e])
            for j in range(dc // 16):
                sl = pl.ds(j * 16, 16)
                rows[e, sl] = rows[e, sl] * nv

        pltpu.sync_copy(rows, acc_sh.at[dst2_v.at[k]], add=True)

    pltpu.async_copy(y_hbm.at[src2_v.at[0]], rows_a, sem_a)

    def pair(g, _):
        k0 = 2 * g
        pltpu.make_async_copy(y_hbm.at[src2_v.at[k0]], rows_a, sem_a).wait()
        pltpu.async_copy(y_hbm.at[src2_v.at[k0 + 1]], rows_b, sem_b)
        scale_scatter(k0, rows_a)
        pltpu.make_async_copy(y_hbm.at[src2_v.at[k0 + 1]], rows_b, sem_b).wait()

        @pl.when(g + 1 < nch // 2)
        def _():
            pltpu.async_copy(y_hbm.at[src2_v.at[k0 + 2]], rows_a, sem_a)

        scale_scatter(k0 + 1, rows_b)
        return 0

    lax.fori_loop(0, nch // 2, pair, 0)
    plsc.subcore_barrier()

    off = 0
    while off < rows_pt:
        sz = min(CH, rows_pt - off)
        pltpu.sync_copy(acc_sh.at[pl.ds(sid * rows_pt + off, sz)],
                        rows_a.at[pl.ds(0, sz)])
        pltpu.sync_copy(rows_a.at[pl.ds(0, sz)],
                        out_hbm.at[pl.ds(cid * n_pad + sid * rows_pt + off, sz)])
        off += sz


def _sc_prop(y, src2, dst2, norm_p):
    n_nodes, dc = y.shape
    n_pad = NS * 640
    ept = norm_p.shape[0] // NW
    assert (ept // CH) % 2 == 0
    mesh = plsc.VectorSubcoreMesh(core_axis_name="c", subcore_axis_name="s")
    kfn = pl.kernel(
        functools.partial(_sc_prop_body, n_pad, dc, ept),
        out_type=jax.ShapeDtypeStruct((NC * n_pad, dc), jnp.float32),
        mesh=mesh,
        compiler_params=pltpu.CompilerParams(needs_layout_passes=False),
        scratch_types=[
            pltpu.VMEM_SHARED((n_pad, dc), jnp.float32),
            pltpu.VMEM((CH, dc), jnp.float32),
            pltpu.VMEM((CH, dc), jnp.float32),
            pltpu.VMEM((ept // CH, CH), jnp.int32),
            pltpu.VMEM((ept // CH, CH), jnp.int32),
            pltpu.VMEM((ept,), jnp.float32),
            pltpu.SemaphoreType.DMA,
            pltpu.SemaphoreType.DMA,
        ],
    )
    return kfn(y, src2, dst2, norm_p)


def _combine_body(a_ref, b_ref, o_ref):
    o_ref[...] = a_ref[...] + b_ref[...]


def _tc_combine(a, b, bm=1000):
    n, d = a.shape
    return pl.pallas_call(
        _combine_body,
        out_shape=jax.ShapeDtypeStruct((n, d), jnp.float32),
        grid=(n // bm,),
        in_specs=[pl.BlockSpec((bm, d), lambda i: (i, 0)),
                  pl.BlockSpec((bm, d), lambda i: (i, 0))],
        out_specs=pl.BlockSpec((bm, d), lambda i: (i, 0)),
    )(a, b)


def _l1pre_body(x_ref, p1_ref, p2a_ref, p2b_ref, w_ref, b_ref, pre_ref, st_ref):
    xc = jnp.concatenate([x_ref[...], p1_ref[...], p2a_ref[...] + p2b_ref[...]],
                         axis=1)
    pre = jnp.dot(xc, w_ref[...], preferred_element_type=jnp.float32) + b_ref[...]
    pre_ref[...] = pre

    @pl.when(pl.program_id(0) == 0)
    def _():
        st_ref[...] = jnp.zeros_like(st_ref)

    st_ref[...] += jnp.stack([jnp.sum(pre, axis=0), jnp.sum(pre * pre, axis=0)])


def _tc_l1pre(x, p1, p2a, p2b, w384, bias1, bm=1000):
    n = x.shape[0]
    f = w384.shape[1]
    return pl.pallas_call(
        _l1pre_body,
        out_shape=[jax.ShapeDtypeStruct((n, f), jnp.float32),
                   jax.ShapeDtypeStruct((2, f), jnp.float32)],
        grid=(n // bm,),
        in_specs=[pl.BlockSpec((bm, 128), lambda i: (i, 0)),
                  pl.BlockSpec((bm, 128), lambda i: (i, 0)),
                  pl.BlockSpec((bm, 128), lambda i: (i, 0)),
                  pl.BlockSpec((bm, 128), lambda i: (i, 0)),
                  pl.BlockSpec((384, f), lambda i: (0, 0)),
                  pl.BlockSpec((f,), lambda i: (0,))],
        out_specs=[pl.BlockSpec((bm, f), lambda i: (i, 0)),
                   pl.BlockSpec((2, f), lambda i: (0, 0))],
    )(x, p1, p2a, p2b, w384, bias1)


def _l1post_body(n_rows, pre_ref, st_ref, g_ref, bb_ref, wu1_ref, wu2_ref,
                 wu3_ref, whb_ref, b2_ref, u1_ref, u2_ref, u3_ref, hb_ref):
    m = st_ref[0, :] / n_rows
    var = st_ref[1, :] / n_rows - m * m
    scale = g_ref[...] * lax.rsqrt(var + 1e-5)
    shift = bb_ref[...] - m * scale
    h = jnp.maximum(pre_ref[...] * scale + shift, 0.0)
    u1_ref[...] = jnp.dot(h, wu1_ref[...], preferred_element_type=jnp.float32)
    u2_ref[...] = jnp.dot(h, wu2_ref[...], preferred_element_type=jnp.float32)
    u3_ref[...] = jnp.dot(h, wu3_ref[...], preferred_element_type=jnp.float32)
    hb_ref[...] = jnp.dot(h, whb_ref[...], preferred_element_type=jnp.float32) + b2_ref[...]


def _tc_l1post(pre, stats, g, b, wu1, wu2, wu3, whb, bias2, bm=1000):
    n, f = pre.shape
    du = wu1.shape[1]
    return pl.pallas_call(
        functools.partial(_l1post_body, float(n)),
        out_shape=[jax.ShapeDtypeStruct((n, du), jnp.float32),
                   jax.ShapeDtypeStruct((n, du), jnp.float32),
                   jax.ShapeDtypeStruct((n, du), jnp.float32),
                   jax.ShapeDtypeStruct((n, 300), jnp.float32)],
        grid=(n // bm,),
        in_specs=[pl.BlockSpec((bm, f), lambda i: (i, 0)),
                  pl.BlockSpec((2, f), lambda i: (0, 0)),
                  pl.BlockSpec((f,), lambda i: (0,)),
                  pl.BlockSpec((f,), lambda i: (0,)),
                  pl.BlockSpec((f, du), lambda i: (0, 0)),
                  pl.BlockSpec((f, du), lambda i: (0, 0)),
                  pl.BlockSpec((f, du), lambda i: (0, 0)),
                  pl.BlockSpec((f, 300), lambda i: (0, 0)),
                  pl.BlockSpec((300,), lambda i: (0,))],
        out_specs=[pl.BlockSpec((bm, du), lambda i: (i, 0)),
                   pl.BlockSpec((bm, du), lambda i: (i, 0)),
                   pl.BlockSpec((bm, du), lambda i: (i, 0)),
                   pl.BlockSpec((bm, 300), lambda i: (i, 0))],
    )(pre, stats, g, b, wu1, wu2, wu3, whb, bias2)


def _fpre_body(hb_ref, v1a_ref, v1b_ref, v2a_ref, v2b_ref, za_ref, zb_ref,
               o_ref, st_ref):
    bm = hb_ref.shape[0]
    v1 = (v1a_ref[...] + v1b_ref[...])[:, :100]
    v23 = (v2a_ref[...] + v2b_ref[...] + 2.0 * (za_ref[...] + zb_ref[...]))[:, :100]
    add = jnp.concatenate([jnp.zeros((bm, 100), jnp.float32), v1, v23], axis=1)
    o = hb_ref[...] + add
    o_ref[...] = o

    @pl.when(pl.program_id(0) == 0)
    def _():
        st_ref[...] = jnp.zeros_like(st_ref)

    st_ref[...] += jnp.stack([jnp.sum(o, axis=0), jnp.sum(o * o, axis=0)])


def _tc_fpre(hb, v1a, v1b, v2a, v2b, za, zb, bm=1000):
    n = hb.shape[0]
    du = v1a.shape[1]
    return pl.pallas_call(
        _fpre_body,
        out_shape=[jax.ShapeDtypeStruct((n, 300), jnp.float32),
                   jax.ShapeDtypeStruct((2, 300), jnp.float32)],
        grid=(n // bm,),
        in_specs=[pl.BlockSpec((bm, 300), lambda i: (i, 0))] +
                 [pl.BlockSpec((bm, du), lambda i: (i, 0))] * 6,
        out_specs=[pl.BlockSpec((bm, 300), lambda i: (i, 0)),
                   pl.BlockSpec((2, 300), lambda i: (0, 0))],
    )(hb, v1a, v1b, v2a, v2b, za, zb)


def _fpost_body(n_rows, o2_ref, st_ref, g_ref, bb_ref, lw_ref, lb_ref, o_ref):
    m = st_ref[0, :] / n_rows
    var = st_ref[1, :] / n_rows - m * m
    scale = g_ref[...] * lax.rsqrt(var + 1e-5)
    shift = bb_ref[...] - m * scale
    h2 = o2_ref[...] * scale + shift
    logits = jnp.dot(h2, lw_ref[...], preferred_element_type=jnp.float32) + lb_ref[...]
    mx = jnp.max(logits, axis=1, keepdims=True)
    lse = jnp.log(jnp.sum(jnp.exp(logits - mx), axis=1, keepdims=True)) + mx
    o_ref[...] = logits - lse


def _tc_fpost(out2, stats, g, b, lw, lb, bm=1000):
    n = out2.shape[0]
    k = lw.shape[1]
    return pl.pallas_call(
        functools.partial(_fpost_body, float(n)),
        out_shape=jax.ShapeDtypeStruct((n, k), jnp.float32),
        grid=(n // bm,),
        in_specs=[pl.BlockSpec((bm, 300), lambda i: (i, 0)),
                  pl.BlockSpec((2, 300), lambda i: (0, 0)),
                  pl.BlockSpec((300,), lambda i: (0,)),
                  pl.BlockSpec((300,), lambda i: (0,)),
                  pl.BlockSpec((300, k), lambda i: (0, 0)),
                  pl.BlockSpec((k,), lambda i: (0,))],
        out_specs=pl.BlockSpec((bm, k), lambda i: (i, 0)),
    )(out2, stats, g, b, lw, lb)


def kernel(x, edge_index, edge_attr, c1s1_W, c1s1_b, c1s2_W, c1s2_b, c1s3_W,
           c1s3_b, c2s1_W, c2s1_b, c2s2_W, c2s2_b, c2s3_W, c2s3_b,
           bn1_g, bn1_b, bn2_g, bn2_b, lin_W, lin_b):
    n, d = x.shape
    e = edge_index.shape[1]
    ept = -(-e // (NW * CH)) * CH
    e_pad = ept * NW

    src = jnp.pad(edge_index[0], (0, e_pad - e))
    dst = jnp.pad(edge_index[1], (0, e_pad - e))
    ew = jnp.pad(edge_attr, (0, e_pad - e))

    n_pad = NS * 640
    src2 = src.reshape(-1, CH)
    dst2 = dst.reshape(-1, CH)
    deg2 = _sc_deg(src, ew, n_pad)
    dis = _tc_dis(deg2, n_pad)
    norm = _sc_norm(src, dst, ew, dis, n_pad)

    p1p = _sc_prop(x, src2, dst2, norm)
    p1 = _tc_combine(p1p[:n], p1p[n_pad:n_pad + n])
    p2p = _sc_prop(p1, src2, dst2, norm)

    z128 = jnp.zeros((128, 400), jnp.float32)
    w384 = jnp.concatenate([
        jnp.concatenate([c1s1_W[0], c1s2_W[0], c1s3_W[0] - c1s3_W[2]], axis=1),
        jnp.concatenate([z128, c1s2_W[1], c1s3_W[1]], axis=1),
        jnp.concatenate([z128, z128, 2.0 * c1s3_W[2]], axis=1)], axis=0)
    bias1 = jnp.concatenate([c1s1_b, c1s2_b, c1s3_b])
    pre, stats1 = _tc_l1pre(x, p1, p2p[:n], p2p[n_pad:n_pad + n], w384, bias1)

    def pad128(w):
        return jnp.pad(w, ((0, 0), (0, 28)))
    whb = jnp.concatenate([c2s1_W[0], c2s2_W[0], c2s3_W[0] - c2s3_W[2]], axis=1)
    bias2 = jnp.concatenate([c2s1_b, c2s2_b, c2s3_b])
    u1, u2, u3, hb = _tc_l1post(pre, stats1, bn1_g, bn1_b,
                                pad128(c2s2_W[1]), pad128(c2s3_W[1]),
                                pad128(c2s3_W[2]), whb, bias2)

    v1p = _sc_prop(u1, src2, dst2, norm)
    v2p = _sc_prop(u2, src2, dst2, norm)
    v3p = _sc_prop(u3, src2, dst2, norm)
    v3 = _tc_combine(v3p[:n], v3p[n_pad:n_pad + n])
    zp = _sc_prop(v3, src2, dst2, norm)

    out2, stats2 = _tc_fpre(hb, v1p[:n], v1p[n_pad:n_pad + n],
                            v2p[:n], v2p[n_pad:n_pad + n],
                            zp[:n], zp[n_pad:n_pad + n])
    return _tc_fpost(out2, stats2, bn2_g, bn2_b, lin_W, lin_b)


# speedup vs baseline: 5.6552x; 1.0934x over previous
import functools
import jax
import jax.numpy as jnp
from jax import lax
from jax.experimental import pallas as pl
from jax.experimental.pallas import tpu as pltpu
from jax.experimental.pallas import tpu_sc as plsc

NC = 2
NS = 16
NW = NC * NS
CH = 128


def _rsqrt16(d):
    ib = plsc.bitcast(d, jnp.int32)
    y = plsc.bitcast(jnp.int32(0x5F3759DF) - (ib >> 1), jnp.float32)
    for _ in range(4):
        y = y * (1.5 - 0.5 * d * y * y)
    return jnp.where(d > 0, y, 0.0)


def _sc_pre_body(n_pad, ept, src2_hbm, dst2_hbm, ew2_hbm, norm_hbm,
                 deg_sh, dis_sh, zbuf, zbuf2, dis_v, srcd_v, ewd_v,
                 srcn_v, dstn_v, ewn_v, nrm_v, sem):
    cid = lax.axis_index("c")
    sid = lax.axis_index("s")
    wid = cid * NS + sid
    rows_pt = n_pad // NS
    nch = ept // CH
    nchd = nch * NC

    for j in range(rows_pt // 16):
        zbuf[pl.ds(j * 16, 16)] = jnp.zeros((16,), jnp.float32)
    pltpu.sync_copy(zbuf, deg_sh.at[pl.ds(sid * rows_pt, rows_pt)])
    plsc.subcore_barrier()

    pltpu.sync_copy(src2_hbm.at[pl.ds(sid * nchd, nchd)], srcd_v)
    pltpu.sync_copy(ew2_hbm.at[pl.ds(sid * nchd, nchd)], ewd_v)

    def fire(k, _):
        pltpu.async_copy(ewd_v.at[k], deg_sh.at[srcd_v.at[k]], sem, add=True)
        return 0

    def drain(k, _):
        pltpu.make_async_copy(ewd_v.at[0], deg_sh.at[srcd_v.at[0]], sem).wait()
        return 0

    lax.fori_loop(0, nchd, fire, 0)
    lax.fori_loop(0, nchd, drain, 0)
    plsc.subcore_barrier()

    pltpu.sync_copy(deg_sh.at[pl.ds(sid * rows_pt, rows_pt)], zbuf)
    for j in range(rows_pt // 16):
        zbuf2[pl.ds(j * 16, 16)] = _rsqrt16(zbuf[pl.ds(j * 16, 16)])
    pltpu.sync_copy(zbuf2, dis_sh.at[pl.ds(sid * rows_pt, rows_pt)])
    plsc.subcore_barrier()
    pltpu.sync_copy(dis_sh, dis_v)

    pltpu.sync_copy(src2_hbm.at[pl.ds(wid * nch, nch)], srcn_v)
    pltpu.sync_copy(dst2_hbm.at[pl.ds(wid * nch, nch)], dstn_v)
    pltpu.sync_copy(ew2_hbm.at[pl.ds(wid * nch, nch)], ewn_v)

    def nchunk(k, _):
        for j in range(CH // 16):
            sl = pl.ds(j * 16, 16)
            a = plsc.load_gather(dis_v, [srcn_v[k, sl]])
            b = plsc.load_gather(dis_v, [dstn_v[k, sl]])
            nrm_v[k, sl] = -(a * ewn_v[k, sl]) * b
        return 0

    lax.fori_loop(0, nch, nchunk, 0)
    pltpu.sync_copy(nrm_v, norm_hbm.at[pl.ds(wid * nch, nch)])


def _sc_pre(src2, dst2, ew2, n_pad):
    nrows = src2.shape[0]
    ept = nrows * CH // NW
    nch = ept // CH
    mesh = plsc.VectorSubcoreMesh(core_axis_name="c", subcore_axis_name="s")
    kfn = pl.kernel(
        functools.partial(_sc_pre_body, n_pad, ept),
        out_type=jax.ShapeDtypeStruct((nrows, CH), jnp.float32),
        mesh=mesh,
        compiler_params=pltpu.CompilerParams(needs_layout_passes=False),
        scratch_types=[
            pltpu.VMEM_SHARED((n_pad,), jnp.float32),
            pltpu.VMEM_SHARED((n_pad,), jnp.float32),
            pltpu.VMEM((n_pad // NS,), jnp.float32),
            pltpu.VMEM((n_pad // NS,), jnp.float32),
            pltpu.VMEM((n_pad,), jnp.float32),
            pltpu.VMEM((nch * NC, CH), jnp.int32),
            pltpu.VMEM((nch * NC, CH), jnp.float32),
            pltpu.VMEM((nch, CH), jnp.int32),
            pltpu.VMEM((nch, CH), jnp.int32),
            pltpu.VMEM((nch, CH), jnp.float32),
            pltpu.VMEM((nch, CH), jnp.float32),
            pltpu.SemaphoreType.DMA,
        ],
    )
    return kfn(src2, dst2, ew2)


def _sc_prop_body(n_pad, dc, ept, bases, y_hbm, src2_hbm, dst2_hbm, nrm2_hbm,
                  out_hbm, acc_sh, rows_a, rows_b, src2_v, dst2_v, nrm_v,
                  shi_a, shi_a2, shi_b, shi_b2, sem_a, sem_a2, sem_b, sem_b2):
    cid = lax.axis_index("c")
    sid = lax.axis_index("s")
    wid = cid * NS + sid
    rows_pt = n_pad // NS
    nch = ept // CH

    pltpu.sync_copy(src2_hbm.at[pl.ds(wid * nch, nch)], src2_v)
    pltpu.sync_copy(dst2_hbm.at[pl.ds(wid * nch, nch)], dst2_v)
    pltpu.sync_copy(nrm2_hbm.at[pl.ds(wid * nch, nch)], nrm_v)

    def zrow(e, _):
        for j in range(dc // 16):
            rows_a[e, pl.ds(j * 16, 16)] = jnp.zeros((16,), jnp.float32)
        return 0

    for p, gb in enumerate(bases):
        dual = len(gb) == 2

        def issue(k, base, rows, shi, sem, add=False):
            if base == 0:
                pltpu.async_copy(y_hbm.at[src2_v.at[k]], rows, sem, add=add)
            else:
                for j in range(CH // 16):
                    sl = pl.ds(j * 16, 16)
                    shi[sl] = src2_v[k, sl] + base
                pltpu.async_copy(y_hbm.at[shi], rows, sem, add=add)

        def wait(rows, sem):
            pltpu.make_async_copy(y_hbm.at[src2_v.at[0]], rows, sem).wait()

        def scale_scatter(k, rows):
            @plsc.parallel_loop(0, CH, unroll=4)
            def scale(e):
                nv = plsc.load_gather(
                    nrm_v, [jnp.zeros((16,), jnp.int32) + k,
                            jnp.zeros((16,), jnp.int32) + e])
                for j in range(dc // 16):
                    sl = pl.ds(j * 16, 16)
                    rows[e, sl] = rows[e, sl] * nv

            pltpu.sync_copy(rows, acc_sh.at[dst2_v.at[k]], add=True)

        lax.fori_loop(0, CH, zrow, 0)
        off = 0
        while off < rows_pt:
            sz = min(CH, rows_pt - off)
            pltpu.sync_copy(rows_a.at[pl.ds(0, sz)],
                            acc_sh.at[pl.ds(sid * rows_pt + off, sz)])
            off += sz
        plsc.subcore_barrier()

        issue(0, gb[0], rows_a, shi_a, sem_a)

        def pair(g, _):
            k0 = 2 * g
            wait(rows_a, sem_a)
            if dual:
                issue(k0, gb[1], rows_a, shi_a2, sem_a2, add=True)
            issue(k0 + 1, gb[0], rows_b, shi_b, sem_b)
            if dual:
                wait(rows_a, sem_a2)
            scale_scatter(k0, rows_a)
            wait(rows_b, sem_b)
            if dual:
                issue(k0 + 1, gb[1], rows_b, shi_b2, sem_b2, add=True)

            @pl.when(g + 1 < nch // 2)
            def _():
                issue(k0 + 2, gb[0], rows_a, shi_a, sem_a)

            if dual:
                wait(rows_b, sem_b2)
            scale_scatter(k0 + 1, rows_b)
            return 0

        lax.fori_loop(0, nch // 2, pair, 0)
        plsc.subcore_barrier()

        orow = (p * NC + cid) * n_pad + sid * rows_pt
        off = 0
        while off < rows_pt:
            sz = min(CH, rows_pt - off)
            pltpu.sync_copy(acc_sh.at[pl.ds(sid * rows_pt + off, sz)],
                            rows_a.at[pl.ds(0, sz)])
            pltpu.sync_copy(rows_a.at[pl.ds(0, sz)],
                            out_hbm.at[pl.ds(orow + off, sz)])
            off += sz


def _sc_prop(y, src2, dst2, norm2, bases=((0,),)):
    n_rows_y, dc = y.shape
    npass = len(bases)
    n_pad = NS * 640
    ept = src2.shape[0] * CH // NW
    nch = ept // CH
    assert nch % 2 == 0
    mesh = plsc.VectorSubcoreMesh(core_axis_name="c", subcore_axis_name="s")
    kfn = pl.kernel(
        functools.partial(_sc_prop_body, n_pad, dc, ept, bases),
        out_type=jax.ShapeDtypeStruct((npass * NC * n_pad, dc), jnp.float32),
        mesh=mesh,
        compiler_params=pltpu.CompilerParams(needs_layout_passes=False),
        scratch_types=[
            pltpu.VMEM_SHARED((n_pad, dc), jnp.float32),
            pltpu.VMEM((CH, dc), jnp.float32),
            pltpu.VMEM((CH, dc), jnp.float32),
            pltpu.VMEM((nch, CH), jnp.int32),
            pltpu.VMEM((nch, CH), jnp.int32),
            pltpu.VMEM((nch, CH), jnp.float32),
            pltpu.VMEM((CH,), jnp.int32),
            pltpu.VMEM((CH,), jnp.int32),
            pltpu.VMEM((CH,), jnp.int32),
            pltpu.VMEM((CH,), jnp.int32),
            pltpu.SemaphoreType.DMA,
            pltpu.SemaphoreType.DMA,
            pltpu.SemaphoreType.DMA,
            pltpu.SemaphoreType.DMA,
        ],
    )
    return kfn(y, src2, dst2, norm2)


def _l1pre_body(x_ref, p1a_ref, p1b_ref, p2a_ref, p2b_ref, w_ref, b_ref,
                pre_ref, st_ref):
    xc = jnp.concatenate([x_ref[...], p1a_ref[...] + p1b_ref[...],
                          p2a_ref[...] + p2b_ref[...]], axis=1)
    pre = jnp.dot(xc, w_ref[...], preferred_element_type=jnp.float32) + b_ref[...]
    pre_ref[...] = pre

    @pl.when(pl.program_id(0) == 0)
    def _():
        st_ref[...] = jnp.zeros_like(st_ref)

    st_ref[...] += jnp.stack([jnp.sum(pre, axis=0), jnp.sum(pre * pre, axis=0)])


def _tc_l1pre(x, p1a, p1b, p2a, p2b, w384, bias1, bm=1000):
    n = x.shape[0]
    f = w384.shape[1]
    return pl.pallas_call(
        _l1pre_body,
        out_shape=[jax.ShapeDtypeStruct((n, f), jnp.float32),
                   jax.ShapeDtypeStruct((2, f), jnp.float32)],
        grid=(n // bm,),
        in_specs=[pl.BlockSpec((bm, 128), lambda i: (i, 0)),
                  pl.BlockSpec((bm, 128), lambda i: (i, 0)),
                  pl.BlockSpec((bm, 128), lambda i: (i, 0)),
                  pl.BlockSpec((bm, 128), lambda i: (i, 0)),
                  pl.BlockSpec((bm, 128), lambda i: (i, 0)),
                  pl.BlockSpec((384, f), lambda i: (0, 0)),
                  pl.BlockSpec((f,), lambda i: (0,))],
        out_specs=[pl.BlockSpec((bm, f), lambda i: (i, 0)),
                   pl.BlockSpec((2, f), lambda i: (0, 0))],
    )(x, p1a, p1b, p2a, p2b, w384, bias1)


def _l1post_body(n_rows, pre_ref, st_ref, g_ref, bb_ref, wu1_ref, wu2_ref,
                 wu3_ref, whb_ref, b2_ref, u1_ref, u2_ref, u3_ref, hb_ref):
    m = st_ref[0, :] / n_rows
    var = st_ref[1, :] / n_rows - m * m
    scale = g_ref[...] * lax.rsqrt(var + 1e-5)
    shift = bb_ref[...] - m * scale
    h = jnp.maximum(pre_ref[...] * scale + shift, 0.0)
    u1_ref[...] = jnp.dot(h, wu1_ref[...], preferred_element_type=jnp.float32)
    u2_ref[...] = jnp.dot(h, wu2_ref[...], preferred_element_type=jnp.float32)
    u3_ref[...] = jnp.dot(h, wu3_ref[...], preferred_element_type=jnp.float32)
    hb_ref[...] = jnp.dot(h, whb_ref[...], preferred_element_type=jnp.float32) + b2_ref[...]


def _tc_l1post(pre, stats, g, b, wu1, wu2, wu3, whb, bias2, bm=1000):
    n, f = pre.shape
    du = wu1.shape[1]
    return pl.pallas_call(
        functools.partial(_l1post_body, float(n)),
        out_shape=[jax.ShapeDtypeStruct((n, du), jnp.float32),
                   jax.ShapeDtypeStruct((n, du), jnp.float32),
                   jax.ShapeDtypeStruct((n, du), jnp.float32),
                   jax.ShapeDtypeStruct((n, 300), jnp.float32)],
        grid=(n // bm,),
        in_specs=[pl.BlockSpec((bm, f), lambda i: (i, 0)),
                  pl.BlockSpec((2, f), lambda i: (0, 0)),
                  pl.BlockSpec((f,), lambda i: (0,)),
                  pl.BlockSpec((f,), lambda i: (0,)),
                  pl.BlockSpec((f, du), lambda i: (0, 0)),
                  pl.BlockSpec((f, du), lambda i: (0, 0)),
                  pl.BlockSpec((f, du), lambda i: (0, 0)),
                  pl.BlockSpec((f, 300), lambda i: (0, 0)),
                  pl.BlockSpec((300,), lambda i: (0,))],
        out_specs=[pl.BlockSpec((bm, du), lambda i: (i, 0)),
                   pl.BlockSpec((bm, du), lambda i: (i, 0)),
                   pl.BlockSpec((bm, du), lambda i: (i, 0)),
                   pl.BlockSpec((bm, 300), lambda i: (i, 0))],
    )(pre, stats, g, b, wu1, wu2, wu3, whb, bias2)


def _fpre_body(hb_ref, v1a_ref, v1b_ref, v2a_ref, v2b_ref, za_ref, zb_ref,
               o_ref, st_ref):
    bm = hb_ref.shape[0]
    v1 = (v1a_ref[...] + v1b_ref[...])[:, :100]
    v23 = (v2a_ref[...] + v2b_ref[...] + 2.0 * (za_ref[...] + zb_ref[...]))[:, :100]
    add = jnp.concatenate([jnp.zeros((bm, 100), jnp.float32), v1, v23], axis=1)
    o = hb_ref[...] + add
    o_ref[...] = o

    @pl.when(pl.program_id(0) == 0)
    def _():
        st_ref[...] = jnp.zeros_like(st_ref)

    st_ref[...] += jnp.stack([jnp.sum(o, axis=0), jnp.sum(o * o, axis=0)])


def _tc_fpre(hb, v1a, v1b, v2a, v2b, za, zb, bm=1000):
    n = hb.shape[0]
    du = v1a.shape[1]
    return pl.pallas_call(
        _fpre_body,
        out_shape=[jax.ShapeDtypeStruct((n, 300), jnp.float32),
                   jax.ShapeDtypeStruct((2, 300), jnp.float32)],
        grid=(n // bm,),
        in_specs=[pl.BlockSpec((bm, 300), lambda i: (i, 0))] +
                 [pl.BlockSpec((bm, du), lambda i: (i, 0))] * 6,
        out_specs=[pl.BlockSpec((bm, 300), lambda i: (i, 0)),
                   pl.BlockSpec((2, 300), lambda i: (0, 0))],
    )(hb, v1a, v1b, v2a, v2b, za, zb)


def _fpost_body(n_rows, o2_ref, st_ref, g_ref, bb_ref, lw_ref, lb_ref, o_ref):
    m = st_ref[0, :] / n_rows
    var = st_ref[1, :] / n_rows - m * m
    scale = g_ref[...] * lax.rsqrt(var + 1e-5)
    shift = bb_ref[...] - m * scale
    h2 = o2_ref[...] * scale + shift
    logits = jnp.dot(h2, lw_ref[...], preferred_element_type=jnp.float32) + lb_ref[...]
    mx = jnp.max(logits, axis=1, keepdims=True)
    lse = jnp.log(jnp.sum(jnp.exp(logits - mx), axis=1, keepdims=True)) + mx
    o_ref[...] = logits - lse


def _tc_fpost(out2, stats, g, b, lw, lb, bm=1000):
    n = out2.shape[0]
    k = lw.shape[1]
    return pl.pallas_call(
        functools.partial(_fpost_body, float(n)),
        out_shape=jax.ShapeDtypeStruct((n, k), jnp.float32),
        grid=(n // bm,),
        in_specs=[pl.BlockSpec((bm, 300), lambda i: (i, 0)),
                  pl.BlockSpec((2, 300), lambda i: (0, 0)),
                  pl.BlockSpec((300,), lambda i: (0,)),
                  pl.BlockSpec((300,), lambda i: (0,)),
                  pl.BlockSpec((300, k), lambda i: (0, 0)),
                  pl.BlockSpec((k,), lambda i: (0,))],
        out_specs=pl.BlockSpec((bm, k), lambda i: (i, 0)),
    )(out2, stats, g, b, lw, lb)


def kernel(x, edge_index, edge_attr, c1s1_W, c1s1_b, c1s2_W, c1s2_b, c1s3_W,
           c1s3_b, c2s1_W, c2s1_b, c2s2_W, c2s2_b, c2s3_W, c2s3_b,
           bn1_g, bn1_b, bn2_g, bn2_b, lin_W, lin_b):
    n, d = x.shape
    e = edge_index.shape[1]
    ept = -(-e // (NW * 2 * CH)) * 2 * CH
    e_pad = ept * NW
    n_pad = NS * 640

    src2 = jnp.pad(edge_index[0], (0, e_pad - e)).reshape(-1, CH)
    dst2 = jnp.pad(edge_index[1], (0, e_pad - e)).reshape(-1, CH)
    ew2 = jnp.pad(edge_attr, (0, e_pad - e)).reshape(-1, CH)

    norm2 = _sc_pre(src2, dst2, ew2, n_pad)

    p1p = _sc_prop(x, src2, dst2, norm2)
    p2p = _sc_prop(p1p, src2, dst2, norm2, ((0, n_pad),))

    z128 = jnp.zeros((128, 400), jnp.float32)
    w384 = jnp.concatenate([
        jnp.concatenate([c1s1_W[0], c1s2_W[0], c1s3_W[0] - c1s3_W[2]], axis=1),
        jnp.concatenate([z128, c1s2_W[1], c1s3_W[1]], axis=1),
        jnp.concatenate([z128, z128, 2.0 * c1s3_W[2]], axis=1)], axis=0)
    bias1 = jnp.concatenate([c1s1_b, c1s2_b, c1s3_b])
    pre, stats1 = _tc_l1pre(x, p1p[:n], p1p[n_pad:n_pad + n],
                            p2p[:n], p2p[n_pad:n_pad + n], w384, bias1)

    def pad128(w):
        return jnp.pad(w, ((0, 0), (0, 28)))
    whb = jnp.concatenate([c2s1_W[0], c2s2_W[0], c2s3_W[0] - c2s3_W[2]], axis=1)
    bias2 = jnp.concatenate([c2s1_b, c2s2_b, c2s3_b])
    u1, u2, u3, hb = _tc_l1post(pre, stats1, bn1_g, bn1_b,
                                pad128(c2s2_W[1]), pad128(c2s3_W[1]),
                                pad128(c2s3_W[2]), whb, bias2)

    u = jnp.concatenate([u1, u2, u3], axis=0)
    up = _sc_prop(u, src2, dst2, norm2, ((0,), (n,), (2 * n,)))
    zp = _sc_prop(up, src2, dst2, norm2, ((4 * n_pad, 5 * n_pad),))

    out2, stats2 = _tc_fpre(hb, up[:n], up[n_pad:n_pad + n],
                            up[2 * n_pad:2 * n_pad + n],
                            up[3 * n_pad:3 * n_pad + n],
                            zp[:n], zp[n_pad:n_pad + n])
    return _tc_fpost(out2, stats2, bn2_g, bn2_b, lin_W, lin_b)


# speedup vs baseline: 5.6939x; 1.0068x over previous
import functools
import jax
import jax.numpy as jnp
from jax import lax
from jax.experimental import pallas as pl
from jax.experimental.pallas import tpu as pltpu
from jax.experimental.pallas import tpu_sc as plsc

NC = 2
NS = 16
NW = NC * NS
CH = 128
_SPLIT = (56, 24)


def _rsqrt16(d):
    ib = plsc.bitcast(d, jnp.int32)
    y = plsc.bitcast(jnp.int32(0x5F3759DF) - (ib >> 1), jnp.float32)
    for _ in range(4):
        y = y * (1.5 - 0.5 * d * y * y)
    return jnp.where(d > 0, y, 0.0)


def _sc_pre_body(n_pad, ept, src2_hbm, dst2_hbm, ew2_hbm, norm_hbm,
                 deg_sh, dis_sh, zbuf, zbuf2, dis_v, srcd_v, ewd_v,
                 srcn_v, dstn_v, ewn_v, nrm_v, sem):
    cid = lax.axis_index("c")
    sid = lax.axis_index("s")
    wid = cid * NS + sid
    rows_pt = n_pad // NS
    nch = ept // CH
    nchd = nch * NC

    for j in range(rows_pt // 16):
        zbuf[pl.ds(j * 16, 16)] = jnp.zeros((16,), jnp.float32)
    pltpu.sync_copy(zbuf, deg_sh.at[pl.ds(sid * rows_pt, rows_pt)])
    plsc.subcore_barrier()

    pltpu.sync_copy(src2_hbm.at[pl.ds(sid * nchd, nchd)], srcd_v)
    pltpu.sync_copy(ew2_hbm.at[pl.ds(sid * nchd, nchd)], ewd_v)

    def fire(k, _):
        pltpu.async_copy(ewd_v.at[k], deg_sh.at[srcd_v.at[k]], sem, add=True)
        return 0

    def drain(k, _):
        pltpu.make_async_copy(ewd_v.at[0], deg_sh.at[srcd_v.at[0]], sem).wait()
        return 0

    lax.fori_loop(0, nchd, fire, 0)
    lax.fori_loop(0, nchd, drain, 0)
    plsc.subcore_barrier()

    pltpu.sync_copy(deg_sh.at[pl.ds(sid * rows_pt, rows_pt)], zbuf)
    for j in range(rows_pt // 16):
        zbuf2[pl.ds(j * 16, 16)] = _rsqrt16(zbuf[pl.ds(j * 16, 16)])
    pltpu.sync_copy(zbuf2, dis_sh.at[pl.ds(sid * rows_pt, rows_pt)])
    plsc.subcore_barrier()
    pltpu.sync_copy(dis_sh, dis_v)

    pltpu.sync_copy(src2_hbm.at[pl.ds(wid * nch, nch)], srcn_v)
    pltpu.sync_copy(dst2_hbm.at[pl.ds(wid * nch, nch)], dstn_v)
    pltpu.sync_copy(ew2_hbm.at[pl.ds(wid * nch, nch)], ewn_v)

    def nchunk(k, _):
        for j in range(CH // 16):
            sl = pl.ds(j * 16, 16)
            a = plsc.load_gather(dis_v, [srcn_v[k, sl]])
            b = plsc.load_gather(dis_v, [dstn_v[k, sl]])
            nrm_v[k, sl] = -(a * ewn_v[k, sl]) * b
        return 0

    lax.fori_loop(0, nch, nchunk, 0)
    pltpu.sync_copy(nrm_v, norm_hbm.at[pl.ds(wid * nch, nch)])


def _sc_pre(src2, dst2, ew2, n_pad, nch_tot):
    nrows = src2.shape[0]
    nch = nch_tot // NW
    ept = nch * CH
    mesh = plsc.VectorSubcoreMesh(core_axis_name="c", subcore_axis_name="s")
    kfn = pl.kernel(
        functools.partial(_sc_pre_body, n_pad, ept),
        out_type=jax.ShapeDtypeStruct((nrows, CH), jnp.float32),
        mesh=mesh,
        compiler_params=pltpu.CompilerParams(needs_layout_passes=False),
        scratch_types=[
            pltpu.VMEM_SHARED((n_pad,), jnp.float32),
            pltpu.VMEM_SHARED((n_pad,), jnp.float32),
            pltpu.VMEM((n_pad // NS,), jnp.float32),
            pltpu.VMEM((n_pad // NS,), jnp.float32),
            pltpu.VMEM((n_pad,), jnp.float32),
            pltpu.VMEM((nch * NC, CH), jnp.int32),
            pltpu.VMEM((nch * NC, CH), jnp.float32),
            pltpu.VMEM((nch, CH), jnp.int32),
            pltpu.VMEM((nch, CH), jnp.int32),
            pltpu.VMEM((nch, CH), jnp.float32),
            pltpu.VMEM((nch, CH), jnp.float32),
            pltpu.SemaphoreType.DMA,
        ],
    )
    return kfn(src2, dst2, ew2)


def _sc_prop_body(n_pad, dc, n0, n1, bases, y_hbm, src2_hbm, dst2_hbm,
                  nrm2_hbm, out_hbm, acc_sh, rows_a, rows_b, src2_v, dst2_v,
                  nrm_v, shi_a, shi_a2, shi_b, shi_b2,
                  sem_a, sem_a2, sem_b, sem_b2):
    cid = lax.axis_index("c")
    sid = lax.axis_index("s")
    rows_pt = n_pad // NS
    row0 = jnp.where(cid == 0, sid * n0, NS * n0 + sid * n1)
    nsc_self = jnp.where(cid == 0, n0 // 8, n1 // 8)

    def zrow(e, _):
        for j in range(dc // 16):
            rows_a[e, pl.ds(j * 16, 16)] = jnp.zeros((16,), jnp.float32)
        return 0

    for p, gb in enumerate(bases):
        dual = len(gb) == 2

        def issue(k, base, rows, shi, sem, add=False):
            if base == 0:
                pltpu.async_copy(y_hbm.at[src2_v.at[k]], rows, sem, add=add)
            else:
                for j in range(CH // 16):
                    sl = pl.ds(j * 16, 16)
                    shi[sl] = src2_v[k, sl] + base
                pltpu.async_copy(y_hbm.at[shi], rows, sem, add=add)

        def wait(rows, sem):
            pltpu.make_async_copy(y_hbm.at[src2_v.at[0]], rows, sem).wait()

        def scale_scatter(k, rows):
            @plsc.parallel_loop(0, CH, unroll=4)
            def scale(e):
                nv = plsc.load_gather(
                    nrm_v, [jnp.zeros((16,), jnp.int32) + k,
                            jnp.zeros((16,), jnp.int32) + e])
                for j in range(dc // 16):
                    sl = pl.ds(j * 16, 16)
                    rows[e, sl] = rows[e, sl] * nv

            pltpu.sync_copy(rows, acc_sh.at[dst2_v.at[k]], add=True)

        lax.fori_loop(0, CH, zrow, 0)
        off = 0
        while off < rows_pt:
            sz = min(CH, rows_pt - off)
            pltpu.sync_copy(rows_a.at[pl.ds(0, sz)],
                            acc_sh.at[pl.ds(sid * rows_pt + off, sz)])
            off += sz
        plsc.subcore_barrier()

        def super_chunk(q, _):
            srow = row0 + q * 8
            pltpu.sync_copy(src2_hbm.at[pl.ds(srow, 8)], src2_v)
            pltpu.sync_copy(dst2_hbm.at[pl.ds(srow, 8)], dst2_v)
            pltpu.sync_copy(nrm2_hbm.at[pl.ds(srow, 8)], nrm_v)
            issue(0, gb[0], rows_a, shi_a, sem_a)
            for gg in range(4):
                k0 = 2 * gg
                wait(rows_a, sem_a)
                if dual:
                    issue(k0, gb[1], rows_a, shi_a2, sem_a2, add=True)
                issue(k0 + 1, gb[0], rows_b, shi_b, sem_b)
                if dual:
                    wait(rows_a, sem_a2)
                scale_scatter(k0, rows_a)
                wait(rows_b, sem_b)
                if dual:
                    issue(k0 + 1, gb[1], rows_b, shi_b2, sem_b2, add=True)
                if gg < 3:
                    issue(k0 + 2, gb[0], rows_a, shi_a, sem_a)
                if dual:
                    wait(rows_b, sem_b2)
                scale_scatter(k0 + 1, rows_b)
            return 0

        lax.fori_loop(0, nsc_self, super_chunk, 0)
        plsc.subcore_barrier()

        orow = (p * NC + cid) * n_pad + sid * rows_pt
        off = 0
        while off < rows_pt:
            sz = min(CH, rows_pt - off)
            pltpu.sync_copy(acc_sh.at[pl.ds(sid * rows_pt + off, sz)],
                            rows_a.at[pl.ds(0, sz)])
            pltpu.sync_copy(rows_a.at[pl.ds(0, sz)],
                            out_hbm.at[pl.ds(orow + off, sz)])
            off += sz


def _sc_prop(y, src2, dst2, norm2, n0, n1, bases=((0,),)):
    n_rows_y, dc = y.shape
    npass = len(bases)
    n_pad = NS * 640
    nmax = max(n0, n1)
    assert n0 % 8 == 0 and n1 % 8 == 0
    mesh = plsc.VectorSubcoreMesh(core_axis_name="c", subcore_axis_name="s")
    kfn = pl.kernel(
        functools.partial(_sc_prop_body, n_pad, dc, n0, n1, bases),
        out_type=jax.ShapeDtypeStruct((npass * NC * n_pad, dc), jnp.float32),
        mesh=mesh,
        compiler_params=pltpu.CompilerParams(needs_layout_passes=False),
        scratch_types=[
            pltpu.VMEM_SHARED((n_pad, dc), jnp.float32),
            pltpu.VMEM((CH, dc), jnp.float32),
            pltpu.VMEM((CH, dc), jnp.float32),
            pltpu.VMEM((8, CH), jnp.int32),
            pltpu.VMEM((8, CH), jnp.int32),
            pltpu.VMEM((8, CH), jnp.float32),
            pltpu.VMEM((CH,), jnp.int32),
            pltpu.VMEM((CH,), jnp.int32),
            pltpu.VMEM((CH,), jnp.int32),
            pltpu.VMEM((CH,), jnp.int32),
            pltpu.SemaphoreType.DMA,
            pltpu.SemaphoreType.DMA,
            pltpu.SemaphoreType.DMA,
            pltpu.SemaphoreType.DMA,
        ],
    )
    return kfn(y, src2, dst2, norm2)


def _l1pre_body(x_ref, p1a_ref, p1b_ref, p2a_ref, p2b_ref, w_ref, b_ref,
                pre_ref, st_ref):
    xc = jnp.concatenate([x_ref[...], p1a_ref[...] + p1b_ref[...],
                          p2a_ref[...] + p2b_ref[...]], axis=1)
    pre = jnp.dot(xc, w_ref[...], preferred_element_type=jnp.float32) + b_ref[...]
    pre_ref[...] = pre

    @pl.when(pl.program_id(0) == 0)
    def _():
        st_ref[...] = jnp.zeros_like(st_ref)

    st_ref[...] += jnp.stack([jnp.sum(pre, axis=0), jnp.sum(pre * pre, axis=0)])


def _tc_l1pre(x, p1a, p1b, p2a, p2b, w384, bias1, bm=1000):
    n = x.shape[0]
    f = w384.shape[1]
    return pl.pallas_call(
        _l1pre_body,
        out_shape=[jax.ShapeDtypeStruct((n, f), jnp.float32),
                   jax.ShapeDtypeStruct((2, f), jnp.float32)],
        grid=(n // bm,),
        in_specs=[pl.BlockSpec((bm, 128), lambda i: (i, 0)),
                  pl.BlockSpec((bm, 128), lambda i: (i, 0)),
                  pl.BlockSpec((bm, 128), lambda i: (i, 0)),
                  pl.BlockSpec((bm, 128), lambda i: (i, 0)),
                  pl.BlockSpec((bm, 128), lambda i: (i, 0)),
                  pl.BlockSpec((384, f), lambda i: (0, 0)),
                  pl.BlockSpec((f,), lambda i: (0,))],
        out_specs=[pl.BlockSpec((bm, f), lambda i: (i, 0)),
                   pl.BlockSpec((2, f), lambda i: (0, 0))],
    )(x, p1a, p1b, p2a, p2b, w384, bias1)


def _l1post_body(n_rows, pre_ref, st_ref, g_ref, bb_ref, wu1_ref, wu2_ref,
                 wu3_ref, whb_ref, b2_ref, u1_ref, u2_ref, u3_ref, hb_ref):
    m = st_ref[0, :] / n_rows
    var = st_ref[1, :] / n_rows - m * m
    scale = g_ref[...] * lax.rsqrt(var + 1e-5)
    shift = bb_ref[...] - m * scale
    h = jnp.maximum(pre_ref[...] * scale + shift, 0.0)
    u1_ref[...] = jnp.dot(h, wu1_ref[...], preferred_element_type=jnp.float32)
    u2_ref[...] = jnp.dot(h, wu2_ref[...], preferred_element_type=jnp.float32)
    u3_ref[...] = jnp.dot(h, wu3_ref[...], preferred_element_type=jnp.float32)
    hb_ref[...] = jnp.dot(h, whb_ref[...], preferred_element_type=jnp.float32) + b2_ref[...]


def _tc_l1post(pre, stats, g, b, wu1, wu2, wu3, whb, bias2, bm=1000):
    n, f = pre.shape
    du = wu1.shape[1]
    return pl.pallas_call(
        functools.partial(_l1post_body, float(n)),
        out_shape=[jax.ShapeDtypeStruct((n, du), jnp.float32),
                   jax.ShapeDtypeStruct((n, du), jnp.float32),
                   jax.ShapeDtypeStruct((n, du), jnp.float32),
                   jax.ShapeDtypeStruct((n, 300), jnp.float32)],
        grid=(n // bm,),
        in_specs=[pl.BlockSpec((bm, f), lambda i: (i, 0)),
                  pl.BlockSpec((2, f), lambda i: (0, 0)),
                  pl.BlockSpec((f,), lambda i: (0,)),
                  pl.BlockSpec((f,), lambda i: (0,)),
                  pl.BlockSpec((f, du), lambda i: (0, 0)),
                  pl.BlockSpec((f, du), lambda i: (0, 0)),
                  pl.BlockSpec((f, du), lambda i: (0, 0)),
                  pl.BlockSpec((f, 300), lambda i: (0, 0)),
                  pl.BlockSpec((300,), lambda i: (0,))],
        out_specs=[pl.BlockSpec((bm, du), lambda i: (i, 0)),
                   pl.BlockSpec((bm, du), lambda i: (i, 0)),
                   pl.BlockSpec((bm, du), lambda i: (i, 0)),
                   pl.BlockSpec((bm, 300), lambda i: (i, 0))],
    )(pre, stats, g, b, wu1, wu2, wu3, whb, bias2)


def _fpre_body(hb_ref, v1a_ref, v1b_ref, v2a_ref, v2b_ref, za_ref, zb_ref,
               o_ref, st_ref):
    bm = hb_ref.shape[0]
    v1 = (v1a_ref[...] + v1b_ref[...])[:, :100]
    v23 = (v2a_ref[...] + v2b_ref[...] + 2.0 * (za_ref[...] + zb_ref[...]))[:, :100]
    add = jnp.concatenate([jnp.zeros((bm, 100), jnp.float32), v1, v23], axis=1)
    o = hb_ref[...] + add
    o_ref[...] = o

    @pl.when(pl.program_id(0) == 0)
    def _():
        st_ref[...] = jnp.zeros_like(st_ref)

    st_ref[...] += jnp.stack([jnp.sum(o, axis=0), jnp.sum(o * o, axis=0)])


def _tc_fpre(hb, v1a, v1b, v2a, v2b, za, zb, bm=1000):
    n = hb.shape[0]
    du = v1a.shape[1]
    return pl.pallas_call(
        _fpre_body,
        out_shape=[jax.ShapeDtypeStruct((n, 300), jnp.float32),
                   jax.ShapeDtypeStruct((2, 300), jnp.float32)],
        grid=(n // bm,),
        in_specs=[pl.BlockSpec((bm, 300), lambda i: (i, 0))] +
                 [pl.BlockSpec((bm, du), lambda i: (i, 0))] * 6,
        out_specs=[pl.BlockSpec((bm, 300), lambda i: (i, 0)),
                   pl.BlockSpec((2, 300), lambda i: (0, 0))],
    )(hb, v1a, v1b, v2a, v2b, za, zb)


def _fpost_body(n_rows, o2_ref, st_ref, g_ref, bb_ref, lw_ref, lb_ref, o_ref):
    m = st_ref[0, :] / n_rows
    var = st_ref[1, :] / n_rows - m * m
    scale = g_ref[...] * lax.rsqrt(var + 1e-5)
    shift = bb_ref[...] - m * scale
    h2 = o2_ref[...] * scale + shift
    logits = jnp.dot(h2, lw_ref[...], preferred_element_type=jnp.float32) + lb_ref[...]
    mx = jnp.max(logits, axis=1, keepdims=True)
    lse = jnp.log(jnp.sum(jnp.exp(logits - mx), axis=1, keepdims=True)) + mx
    o_ref[...] = logits - lse


def _tc_fpost(out2, stats, g, b, lw, lb, bm=1000):
    n = out2.shape[0]
    k = lw.shape[1]
    return pl.pallas_call(
        functools.partial(_fpost_body, float(n)),
        out_shape=jax.ShapeDtypeStruct((n, k), jnp.float32),
        grid=(n // bm,),
        in_specs=[pl.BlockSpec((bm, 300), lambda i: (i, 0)),
                  pl.BlockSpec((2, 300), lambda i: (0, 0)),
                  pl.BlockSpec((300,), lambda i: (0,)),
                  pl.BlockSpec((300,), lambda i: (0,)),
                  pl.BlockSpec((300, k), lambda i: (0, 0)),
                  pl.BlockSpec((k,), lambda i: (0,))],
        out_specs=pl.BlockSpec((bm, k), lambda i: (i, 0)),
    )(out2, stats, g, b, lw, lb)


def kernel(x, edge_index, edge_attr, c1s1_W, c1s1_b, c1s2_W, c1s2_b, c1s3_W,
           c1s3_b, c2s1_W, c2s1_b, c2s2_W, c2s2_b, c2s3_W, c2s3_b,
           bn1_g, bn1_b, bn2_g, bn2_b, lin_W, lin_b):
    n, d = x.shape
    e = edge_index.shape[1]
    ept = -(-e // (NW * 2 * CH)) * 2 * CH
    e_pad = ept * NW
    nch_tot = e_pad // CH
    n_pad = NS * 640

    n0, n1 = _SPLIT
    assert (n0 + n1) * NS == nch_tot
    nmax = max(n0, n1)
    xtr = nmax * CH
    src2 = jnp.pad(edge_index[0], (0, e_pad + xtr - e)).reshape(-1, CH)
    dst2 = jnp.pad(edge_index[1], (0, e_pad + xtr - e)).reshape(-1, CH)
    ew2 = jnp.pad(edge_attr, (0, e_pad + xtr - e)).reshape(-1, CH)

    norm2 = _sc_pre(src2, dst2, ew2, n_pad, nch_tot)

    p1p = _sc_prop(x, src2, dst2, norm2, n0, n1)
    p2p = _sc_prop(p1p, src2, dst2, norm2, n0, n1, ((0, n_pad),))

    z128 = jnp.zeros((128, 400), jnp.float32)
    w384 = jnp.concatenate([
        jnp.concatenate([c1s1_W[0], c1s2_W[0], c1s3_W[0] - c1s3_W[2]], axis=1),
        jnp.concatenate([z128, c1s2_W[1], c1s3_W[1]], axis=1),
        jnp.concatenate([z128, z128, 2.0 * c1s3_W[2]], axis=1)], axis=0)
    bias1 = jnp.concatenate([c1s1_b, c1s2_b, c1s3_b])
    pre, stats1 = _tc_l1pre(x, p1p[:n], p1p[n_pad:n_pad + n],
                            p2p[:n], p2p[n_pad:n_pad + n], w384, bias1)

    def pad128(w):
        return jnp.pad(w, ((0, 0), (0, 28)))
    whb = jnp.concatenate([c2s1_W[0], c2s2_W[0], c2s3_W[0] - c2s3_W[2]], axis=1)
    bias2 = jnp.concatenate([c2s1_b, c2s2_b, c2s3_b])
    u1, u2, u3, hb = _tc_l1post(pre, stats1, bn1_g, bn1_b,
                                pad128(c2s2_W[1]), pad128(c2s3_W[1]),
                                pad128(c2s3_W[2]), whb, bias2)

    u = jnp.concatenate([u1, u2, u3], axis=0)
    up = _sc_prop(u, src2, dst2, norm2, n0, n1, ((0,), (n,), (2 * n,)))
    zp = _sc_prop(up, src2, dst2, norm2, n0, n1, ((4 * n_pad, 5 * n_pad),))

    out2, stats2 = _tc_fpre(hb, up[:n], up[n_pad:n_pad + n],
                            up[2 * n_pad:2 * n_pad + n],
                            up[3 * n_pad:3 * n_pad + n],
                            zp[:n], zp[n_pad:n_pad + n])
    return _tc_fpost(out2, stats2, bn2_g, bn2_b, lin_W, lin_b)
